# Initial kernel scaffold; baseline (speedup 1.0000x reference)
import jax
import jax.numpy as jnp
from jax.experimental import pallas as pl


def kernel(feat, edge_index, W1, b1, W2, b2):
    raise NotImplementedError("write your pallas kernel here")


# speedup vs baseline: 4.9577x; 4.9577x over previous
import functools

import jax
import jax.numpy as jnp
from jax import lax
from jax.experimental import pallas as pl
from jax.experimental.pallas import tpu as pltpu
from jax.experimental.pallas import tpu_sc as plsc

N = 10000
NP = 10112
E = 320000
D = 32
NC = 2
NS = 16
NW = NC * NS
SUB = 128
NSUB = 8
C = SUB * NSUB
CHUNKS = 10
EPT = C * CHUNKS
EP = EPT * NW
RPT = NP // NS
GAMMA = 0.25
BETA = 2.0
LAM = 3.0

F32 = jnp.float32
I32 = jnp.int32

_SC_PARAMS = pltpu.CompilerParams(use_tc_tiling_on_sc=False,
                                  needs_layout_passes=False)


def _mesh():
    return plsc.VectorSubcoreMesh(core_axis_name="c", subcore_axis_name="s",
                                  num_cores=NC, num_subcores=NS)


def _fill(ref, rows, width, value):
    v = jnp.full((16,), value, F32)

    def body(i, carry):
        for k in range(width // 16):
            ref[i, pl.ds(k * 16, 16)] = v
        return carry

    lax.fori_loop(0, rows, body, 0)


def _nrsqrt(s):
    i = plsc.bitcast(s, I32)
    i = jnp.int32(0x5F3759DF) - lax.shift_right_logical(i, 1)
    y = plsc.bitcast(i, F32)
    for _ in range(3):
        y = y * (1.5 - 0.5 * s * y * y)
    return y


def _tile_ids():
    c = lax.axis_index("c")
    s = lax.axis_index("s")
    return c, s, c * NS + s


def _deg_kernel():
    scratch = [
        pltpu.VMEM((NSUB, SUB), I32),
        pltpu.VMEM((NSUB, SUB), I32),
        pltpu.VMEM((SUB, 16), F32),
        pltpu.VMEM((RPT, 16), F32),
        pltpu.VMEM_SHARED((NP, 16), F32),
        pltpu.VMEM_SHARED((NP, 16), F32),
    ]
    out_type = (jax.ShapeDtypeStruct((NC, NP, 16), F32),
                jax.ShapeDtypeStruct((NC, NP, 16), F32))

    def body(src_h, dst_h, do_out, di_out, idx_s, idx_d, ones_b, zstage,
             do_sh, di_sh):
        c, s, wid = _tile_ids()
        _fill(ones_b, SUB, 16, 1.0)
        _fill(zstage, RPT, 16, 0.0)
        pltpu.sync_copy(zstage, do_sh.at[pl.ds(s * RPT, RPT)])
        pltpu.sync_copy(zstage, di_sh.at[pl.ds(s * RPT, RPT)])
        plsc.subcore_barrier()

        def chunk(ch, carry):
            rb = wid * (EPT // SUB) + ch * NSUB
            pltpu.sync_copy(src_h.at[pl.ds(rb, NSUB)], idx_s)
            pltpu.sync_copy(dst_h.at[pl.ds(rb, NSUB)], idx_d)
            for j in range(NSUB):
                pltpu.sync_copy(ones_b, do_sh.at[idx_s.at[j]], add=True)
                pltpu.sync_copy(ones_b, di_sh.at[idx_d.at[j]], add=True)
            return carry

        lax.fori_loop(0, CHUNKS, chunk, 0)
        plsc.subcore_barrier()
        rb = s * RPT
        pltpu.sync_copy(do_sh.at[pl.ds(rb, RPT)], do_out.at[c, pl.ds(rb, RPT)])
        pltpu.sync_copy(di_sh.at[pl.ds(rb, RPT)], di_out.at[c, pl.ds(rb, RPT)])

    return pl.kernel(body, out_type=out_type, mesh=_mesh(),
                     scratch_types=scratch, name="sc_degrees",
                     compiler_params=_SC_PARAMS)


def _pass_a_kernel():
    scratch = [
        pltpu.VMEM((NSUB, SUB), I32),
        pltpu.VMEM((NSUB, SUB), I32),
        pltpu.VMEM((C, D), F32),
        pltpu.VMEM_SHARED((NP, D), F32),
    ]
    out_type = jax.ShapeDtypeStruct((NC, NP, D), F32)

    def body(xs_h, src_h, dst_h, a_out, idx_s, idx_d, rows, a_sh):
        c, s, wid = _tile_ids()
        _fill(rows, RPT, D, 0.0)
        pltpu.sync_copy(rows.at[pl.ds(0, RPT)], a_sh.at[pl.ds(s * RPT, RPT)])
        plsc.subcore_barrier()

        def chunk(ch, carry):
            rb = wid * (EPT // SUB) + ch * NSUB
            pltpu.sync_copy(src_h.at[pl.ds(rb, NSUB)], idx_s)
            pltpu.sync_copy(dst_h.at[pl.ds(rb, NSUB)], idx_d)
            for j in range(NSUB):
                pltpu.sync_copy(xs_h.at[idx_s.at[j]],
                                rows.at[pl.ds(j * SUB, SUB)])
            for j in range(NSUB):
                pltpu.sync_copy(rows.at[pl.ds(j * SUB, SUB)],
                                a_sh.at[idx_d.at[j]], add=True)
            return carry

        lax.fori_loop(0, CHUNKS, chunk, 0)
        plsc.subcore_barrier()
        rb = s * RPT
        pltpu.sync_copy(a_sh.at[pl.ds(rb, RPT)], a_out.at[c, pl.ds(rb, RPT)])

    return pl.kernel(body, out_type=out_type, mesh=_mesh(),
                     scratch_types=scratch, name="sc_adj",
                     compiler_params=_SC_PARAMS)


def _pass_b_kernel(read_z, write_z):
    out_type = []
    if write_z:
        out_type.append(jax.ShapeDtypeStruct((EP, D), F32))
    out_type.append(jax.ShapeDtypeStruct((NC, NP, D), F32))
    out_type.append(jax.ShapeDtypeStruct((NC, NP, D), F32))
    CH = C // 2
    scratch = [
        pltpu.VMEM((NSUB, SUB), I32),
        pltpu.VMEM((NSUB, SUB), I32),
        pltpu.VMEM((CH, D), F32),
        pltpu.VMEM((CH, D), F32),
        pltpu.VMEM((CH, D), F32),
        pltpu.VMEM_SHARED((NP, D), F32),
        pltpu.VMEM_SHARED((NP, D), F32),
    ]

    def body(*refs):
        i = 4
        xbs_h, xbd_h, src_h, dst_h = refs[:4]
        z_in = refs[i] if read_z else None
        i += 1 if read_z else 0
        z_out = refs[i] if write_z else None
        i += 1 if write_z else 0
        p_out, q_out = refs[i], refs[i + 1]
        idx_s, idx_d, rows_a, rows_b, zbuf, p_sh, q_sh = refs[i + 2:]
        CH = C // 2

        c, s, wid = _tile_ids()
        _fill(zbuf, CH, D, 0.0)
        rb0 = s * RPT
        pltpu.sync_copy(zbuf, p_sh.at[pl.ds(rb0, CH)])
        pltpu.sync_copy(zbuf.at[pl.ds(0, RPT - CH)],
                        p_sh.at[pl.ds(rb0 + CH, RPT - CH)])
        pltpu.sync_copy(zbuf, q_sh.at[pl.ds(rb0, CH)])
        pltpu.sync_copy(zbuf.at[pl.ds(0, RPT - CH)],
                        q_sh.at[pl.ds(rb0 + CH, RPT - CH)])
        plsc.subcore_barrier()

        iota = lax.iota(I32, 16)

        def chunk(ch, carry):
            rb = wid * (EPT // SUB) + ch * NSUB
            pltpu.sync_copy(src_h.at[pl.ds(rb, NSUB)], idx_s)
            pltpu.sync_copy(dst_h.at[pl.ds(rb, NSUB)], idx_d)
            for h in range(2):
                eb = wid * EPT + ch * C + h * CH
                for j in range(NSUB // 2):
                    jj = h * (NSUB // 2) + j
                    pltpu.sync_copy(xbs_h.at[idx_s.at[jj]],
                                    rows_a.at[pl.ds(j * SUB, SUB)])
                    pltpu.sync_copy(xbd_h.at[idx_d.at[jj]],
                                    rows_b.at[pl.ds(j * SUB, SUB)])
                if read_z:
                    pltpu.sync_copy(z_in.at[pl.ds(eb, CH)], zbuf)

                def group(g, gcarry):
                    row = g * 16 + iota
                    ssum = jnp.zeros((16,), F32)
                    zcols = []
                    for j in range(D):
                        cj = jnp.full((16,), j, I32)
                        a = plsc.load_gather(rows_a, [row, cj])
                        b = plsc.load_gather(rows_b, [row, cj])
                        zb = BETA * (a - b)
                        if read_z:
                            zb = zb + plsc.load_gather(zbuf, [row, cj])
                        ssum = ssum + zb * zb
                        zcols.append(zb)
                    over = ssum > LAM * LAM
                    r = _nrsqrt(jnp.maximum(ssum, LAM * LAM))
                    scale = jnp.where(over, LAM * r, 1.0)
                    for j in range(D):
                        cj = jnp.full((16,), j, I32)
                        plsc.store_scatter(zbuf, [row, cj], zcols[j] * scale)
                    return gcarry

                lax.fori_loop(0, CH // 16, group, 0)

                if write_z:
                    pltpu.sync_copy(zbuf, z_out.at[pl.ds(eb, CH)])
                for j in range(NSUB // 2):
                    jj = h * (NSUB // 2) + j
                    pltpu.sync_copy(zbuf.at[pl.ds(j * SUB, SUB)],
                                    p_sh.at[idx_s.at[jj]], add=True)
                    pltpu.sync_copy(zbuf.at[pl.ds(j * SUB, SUB)],
                                    q_sh.at[idx_d.at[jj]], add=True)
            return carry

        lax.fori_loop(0, CHUNKS, chunk, 0)
        plsc.subcore_barrier()
        rb = s * RPT
        pltpu.sync_copy(p_sh.at[pl.ds(rb, RPT)], p_out.at[c, pl.ds(rb, RPT)])
        pltpu.sync_copy(q_sh.at[pl.ds(rb, RPT)], q_out.at[c, pl.ds(rb, RPT)])

    return pl.kernel(body, out_type=tuple(out_type), mesh=_mesh(),
                     scratch_types=scratch,
                     name=f"sc_dual_r{int(read_z)}w{int(write_z)}",
                     compiler_params=_SC_PARAMS)


def _mlp(feat, W1, b1, W2, b2):
    def body(f_ref, w1_ref, b1_ref, w2_ref, b2_ref, o_ref):
        h1 = jnp.dot(f_ref[...], w1_ref[...], preferred_element_type=F32)
        h1 = jnp.maximum(h1 + b1_ref[...], 0.0)
        o_ref[...] = jnp.dot(h1, w2_ref[...],
                             preferred_element_type=F32) + b2_ref[...]

    return pl.pallas_call(
        body,
        grid=(10,),
        in_specs=[
            pl.BlockSpec((1000, 128), lambda i: (i, 0)),
            pl.BlockSpec((128, 64), lambda i: (0, 0)),
            pl.BlockSpec((1, 64), lambda i: (0, 0)),
            pl.BlockSpec((64, 32), lambda i: (0, 0)),
            pl.BlockSpec((1, 32), lambda i: (0, 0)),
        ],
        out_specs=pl.BlockSpec((1000, 32), lambda i: (i, 0)),
        out_shape=jax.ShapeDtypeStruct((N, D), F32),
    )(feat, W1, b1.reshape(1, 64), W2, b2.reshape(1, 32))


_NB = 2528


def _nspec(shape3=False, width=D):
    if shape3:
        return pl.BlockSpec((NC, _NB, width), lambda i: (0, i, 0))
    return pl.BlockSpec((_NB, width), lambda i: (i, 0))


def _nshape():
    return jax.ShapeDtypeStruct((NP, D), F32)


def _prep(dego, degi, h_pad):
    def body(do_ref, di_ref, h_ref, dob_ref, dib_ref, xs_ref):
        dso = do_ref[0, :, 0:1] + do_ref[1, :, 0:1]
        dsi = di_ref[0, :, 0:1] + di_ref[1, :, 0:1]
        dob = jnp.broadcast_to(lax.rsqrt(jnp.maximum(dso, 1.0)), (_NB, D))
        dib = jnp.broadcast_to(lax.rsqrt(jnp.maximum(dsi, 1.0)), (_NB, D))
        dob_ref[...] = dob
        dib_ref[...] = dib
        xs_ref[...] = dob * h_ref[...]

    return pl.pallas_call(
        body,
        grid=(NP // _NB,),
        in_specs=[_nspec(True, 16), _nspec(True, 16), _nspec()],
        out_specs=[_nspec(), _nspec(), _nspec()],
        out_shape=[_nshape(), _nshape(), _nshape()],
    )(dego, degi, h_pad)


def _node1(A, h_pad, dob, dib, P=None, Q=None):
    have_pq = P is not None

    def body(*refs):
        a_ref, h_ref, dob_ref, dib_ref = refs[:4]
        i = 4
        if have_pq:
            p_ref, q_ref = refs[i], refs[i + 1]
            i += 2
        y_ref, xbs_ref, xbd_ref = refs[i:]
        dob_v, dib_v = dob_ref[...], dib_ref[...]
        y = GAMMA * h_ref[...] + (1.0 - GAMMA) * dib_v * (
            a_ref[0] + a_ref[1])
        xbar = y
        if have_pq:
            u = dob_v * (p_ref[0] + p_ref[1]) - dib_v * (q_ref[0] + q_ref[1])
            xbar = y - GAMMA * u
        y_ref[...] = y
        xbs_ref[...] = dob_v * xbar
        xbd_ref[...] = dib_v * xbar

    in_specs = [_nspec(True), _nspec(), _nspec(), _nspec()]
    args = [A, h_pad, dob, dib]
    if have_pq:
        in_specs += [_nspec(True), _nspec(True)]
        args += [P, Q]
    return pl.pallas_call(
        body,
        grid=(NP // _NB,),
        in_specs=in_specs,
        out_specs=[_nspec(), _nspec(), _nspec()],
        out_shape=[_nshape(), _nshape(), _nshape()],
    )(*args)


def _node2(y, P, Q, dob, dib, want_xs):
    def body(*refs):
        y_ref, p_ref, q_ref, dob_ref, dib_ref = refs[:5]
        outs = refs[5:]
        dob_v, dib_v = dob_ref[...], dib_ref[...]
        u = dob_v * (p_ref[0] + p_ref[1]) - dib_v * (q_ref[0] + q_ref[1])
        x = y_ref[...] - GAMMA * u
        outs[0][...] = x
        if want_xs:
            outs[1][...] = dob_v * x

    n_out = 2 if want_xs else 1
    return pl.pallas_call(
        body,
        grid=(NP // _NB,),
        in_specs=[_nspec(), _nspec(True), _nspec(True), _nspec(), _nspec()],
        out_specs=[_nspec()] * n_out,
        out_shape=[_nshape()] * n_out,
    )(y, P, Q, dob, dib)


@jax.jit
def kernel(feat, edge_index, W1, b1, W2, b2):
    src = edge_index[0]
    dst = edge_index[1]
    pad = jnp.full((EP - E,), N, I32)
    src_p = jnp.concatenate([src, pad]).reshape(EP // SUB, SUB)
    dst_p = jnp.concatenate([dst, pad]).reshape(EP // SUB, SUB)

    h = _mlp(feat, W1, b1, W2, b2)
    h_pad = jnp.pad(h, ((0, NP - N), (0, 0)))

    dego, degi = _deg_kernel()(src_p, dst_p)
    dob, dib, xs = _prep(dego, degi, h_pad)

    pass_a = _pass_a_kernel()
    pass_b = _pass_b_kernel(read_z=True, write_z=True)
    z = jnp.zeros((EP, D), F32)

    A = pass_a(xs, src_p, dst_p)
    y, xbs, xbd = _node1(A, h_pad, dob, dib)
    z, P, Q = pass_b(xbs, xbd, src_p, dst_p, z)
    x, xs = _node2(y, P, Q, dob, dib, want_xs=True)

    A = pass_a(xs, src_p, dst_p)
    y, xbs, xbd = _node1(A, h_pad, dob, dib, P, Q)
    z, P, Q = pass_b(xbs, xbd, src_p, dst_p, z)
    x, xs = _node2(y, P, Q, dob, dib, want_xs=True)

    A = pass_a(xs, src_p, dst_p)
    y, xbs, xbd = _node1(A, h_pad, dob, dib, P, Q)
    z, P, Q = pass_b(xbs, xbd, src_p, dst_p, z)
    (x,) = _node2(y, P, Q, dob, dib, want_xs=False)

    return x[:N]


# speedup vs baseline: 5.1629x; 1.0414x over previous
import jax
import jax.numpy as jnp
from jax import lax
from jax.experimental import pallas as pl
from jax.experimental.pallas import tpu as pltpu
from jax.experimental.pallas import tpu_sc as plsc

N = 10000
NP = 10112
E = 320000
D = 32
NC = 2
NS = 16
NW = NC * NS
SUB = 128
EPT = 10240
EP = EPT * NW
IROWS = EPT // SUB
RPT = NP // NS
GAMMA = 0.25
BETA = 2.0
LAM = 3.0

F32 = jnp.float32
I32 = jnp.int32

_SC_PARAMS = pltpu.CompilerParams(use_tc_tiling_on_sc=False,
                                  needs_layout_passes=False)


def _mesh():
    return plsc.VectorSubcoreMesh(core_axis_name="c", subcore_axis_name="s",
                                  num_cores=NC, num_subcores=NS)


def _fill(ref, rows, width, value):
    v = jnp.full((16,), value, F32)

    def body(i, carry):
        for k in range(width // 16):
            ref[i, pl.ds(k * 16, 16)] = v
        return carry

    lax.fori_loop(0, rows, body, 0)


def _zero_acc(zsrc, acc_sh, s, ch):
    base = s * RPT
    off = 0
    while off < RPT:
        ln = min(ch, RPT - off)
        pltpu.sync_copy(zsrc.at[pl.ds(0, ln)], acc_sh.at[pl.ds(base + off, ln)])
        off += ln


def _nrsqrt(s):
    i = plsc.bitcast(s, I32)
    i = jnp.int32(0x5F3759DF) - lax.shift_right_logical(i, 1)
    y = plsc.bitcast(i, F32)
    for _ in range(3):
        y = y * (1.5 - 0.5 * s * y * y)
    return y


def _tile_ids():
    c = lax.axis_index("c")
    s = lax.axis_index("s")
    return c, s, c * NS + s


def _deg_kernel():
    NSUB = 8
    scratch = [
        pltpu.VMEM((NSUB, SUB), I32),
        pltpu.VMEM((NSUB, SUB), I32),
        pltpu.VMEM((SUB, 16), F32),
        pltpu.VMEM((RPT, 16), F32),
        pltpu.VMEM_SHARED((NP, 16), F32),
        pltpu.VMEM_SHARED((NP, 16), F32),
    ]
    out_type = (jax.ShapeDtypeStruct((NC, NP, 16), F32),
                jax.ShapeDtypeStruct((NC, NP, 16), F32))

    def body(src_h, dst_h, do_out, di_out, idx_s, idx_d, ones_b, zstage,
             do_sh, di_sh):
        c, s, wid = _tile_ids()
        _fill(ones_b, SUB, 16, 1.0)
        _fill(zstage, RPT, 16, 0.0)
        pltpu.sync_copy(zstage, do_sh.at[pl.ds(s * RPT, RPT)])
        pltpu.sync_copy(zstage, di_sh.at[pl.ds(s * RPT, RPT)])
        plsc.subcore_barrier()

        def chunk(ch, carry):
            rb = wid * IROWS + ch * NSUB
            pltpu.sync_copy(src_h.at[pl.ds(rb, NSUB)], idx_s)
            pltpu.sync_copy(dst_h.at[pl.ds(rb, NSUB)], idx_d)
            for j in range(NSUB):
                pltpu.sync_copy(ones_b, do_sh.at[idx_s.at[j]], add=True)
                pltpu.sync_copy(ones_b, di_sh.at[idx_d.at[j]], add=True)
            return carry

        lax.fori_loop(0, IROWS // NSUB, chunk, 0)
        plsc.subcore_barrier()
        rb = s * RPT
        pltpu.sync_copy(do_sh.at[pl.ds(rb, RPT)], do_out.at[c, pl.ds(rb, RPT)])
        pltpu.sync_copy(di_sh.at[pl.ds(rb, RPT)], di_out.at[c, pl.ds(rb, RPT)])

    return pl.kernel(body, out_type=out_type, mesh=_mesh(),
                     scratch_types=scratch, name="sc_degrees",
                     compiler_params=_SC_PARAMS)


def _pass_a_kernel():
    CH = 512
    TPS = CH // SUB
    H = EPT // CH
    scratch = [
        pltpu.VMEM((IROWS, SUB), I32),
        pltpu.VMEM((IROWS, SUB), I32),
        pltpu.VMEM((CH, D), F32),
        pltpu.VMEM((CH, D), F32),
        pltpu.VMEM_SHARED((NP, D), F32),
        pltpu.SemaphoreType.DMA,
        pltpu.SemaphoreType.DMA,
        pltpu.SemaphoreType.DMA,
        pltpu.SemaphoreType.DMA,
    ]
    out_type = jax.ShapeDtypeStruct((NC, NP, D), F32)

    def body(xs_h, src_h, dst_h, a_out, idx_s, idx_d, r0, r1, a_sh,
             g0, g1, s0, s1):
        ROWS = (r0, r1)
        GS = (g0, g1)
        SS = (s0, s1)
        c, s, wid = _tile_ids()
        pltpu.sync_copy(src_h.at[pl.ds(wid * IROWS, IROWS)], idx_s)
        pltpu.sync_copy(dst_h.at[pl.ds(wid * IROWS, IROWS)], idx_d)
        _fill(r0, CH, D, 0.0)
        _zero_acc(r0, a_sh, s, CH)
        plsc.subcore_barrier()

        def gathers(t, p, fire):
            for u in range(TPS):
                cp = pltpu.make_async_copy(
                    xs_h.at[idx_s.at[t * TPS + u]],
                    ROWS[p].at[pl.ds(u * SUB, SUB)], GS[p])
                cp.start() if fire else cp.wait()

        def scatters(t, p, fire):
            for u in range(TPS):
                cp = pltpu.make_async_copy(
                    ROWS[p].at[pl.ds(u * SUB, SUB)],
                    a_sh.at[idx_d.at[t * TPS + u]], SS[p])
                cp.start(add=True) if fire else cp.wait()

        gathers(0, 0, True)
        gathers(1, 1, True)

        def step(k, carry):
            t0 = 2 * k
            gathers(t0, 0, False)
            scatters(t0, 0, True)
            gathers(t0 + 1, 1, False)
            scatters(t0 + 1, 1, True)
            scatters(t0, 0, False)
            gathers(t0 + 2, 0, True)
            scatters(t0 + 1, 1, False)
            gathers(t0 + 3, 1, True)
            return carry

        lax.fori_loop(0, H // 2 - 1, step, 0)
        gathers(H - 2, 0, False)
        scatters(H - 2, 0, True)
        gathers(H - 1, 1, False)
        scatters(H - 1, 1, True)
        scatters(H - 2, 0, False)
        scatters(H - 1, 1, False)
        plsc.subcore_barrier()
        rb = s * RPT
        pltpu.sync_copy(a_sh.at[pl.ds(rb, RPT)], a_out.at[c, pl.ds(rb, RPT)])

    return pl.kernel(body, out_type=out_type, mesh=_mesh(),
                     scratch_types=scratch, name="sc_adj",
                     compiler_params=_SC_PARAMS)


def _pass_b_kernel():
    read_z = True
    write_z = True
    NSUB = 8
    C = 1024
    CHUNKS = 10
    out_type = []
    if write_z:
        out_type.append(jax.ShapeDtypeStruct((EP, D), F32))
    out_type.append(jax.ShapeDtypeStruct((NC, NP, D), F32))
    out_type.append(jax.ShapeDtypeStruct((NC, NP, D), F32))
    CH = C // 2
    scratch = [
        pltpu.VMEM((NSUB, SUB), I32),
        pltpu.VMEM((NSUB, SUB), I32),
        pltpu.VMEM((CH, D), F32),
        pltpu.VMEM((CH, D), F32),
        pltpu.VMEM((CH, D), F32),
        pltpu.VMEM_SHARED((NP, D), F32),
        pltpu.VMEM_SHARED((NP, D), F32),
    ]

    def body(*refs):
        i = 4
        xbs_h, xbd_h, src_h, dst_h = refs[:4]
        z_in = refs[i] if read_z else None
        i += 1 if read_z else 0
        z_out = refs[i] if write_z else None
        i += 1 if write_z else 0
        p_out, q_out = refs[i], refs[i + 1]
        idx_s, idx_d, rows_a, rows_b, zbuf, p_sh, q_sh = refs[i + 2:]
        CH = C // 2

        c, s, wid = _tile_ids()
        _fill(zbuf, CH, D, 0.0)
        _zero_acc(zbuf, p_sh, s, CH)
        _zero_acc(zbuf, q_sh, s, CH)
        plsc.subcore_barrier()

        iota = lax.iota(I32, 16)

        def chunk(ch, carry):
            rb = wid * IROWS + ch * NSUB
            pltpu.sync_copy(src_h.at[pl.ds(rb, NSUB)], idx_s)
            pltpu.sync_copy(dst_h.at[pl.ds(rb, NSUB)], idx_d)
            for h in range(2):
                eb = wid * EPT + ch * C + h * CH
                for j in range(NSUB // 2):
                    jj = h * (NSUB // 2) + j
                    pltpu.sync_copy(xbs_h.at[idx_s.at[jj]],
                                    rows_a.at[pl.ds(j * SUB, SUB)])
                    pltpu.sync_copy(xbd_h.at[idx_d.at[jj]],
                                    rows_b.at[pl.ds(j * SUB, SUB)])
                if read_z:
                    pltpu.sync_copy(z_in.at[pl.ds(eb, CH)], zbuf)

                def group(g, gcarry):
                    row = g * 16 + iota
                    ssum = jnp.zeros((16,), F32)
                    zcols = []
                    for j in range(D):
                        cj = jnp.full((16,), j, I32)
                        a = plsc.load_gather(rows_a, [row, cj])
                        b = plsc.load_gather(rows_b, [row, cj])
                        zb = BETA * (a - b)
                        if read_z:
                            zb = zb + plsc.load_gather(zbuf, [row, cj])
                        ssum = ssum + zb * zb
                        zcols.append(zb)
                    over = ssum > LAM * LAM
                    r = _nrsqrt(jnp.maximum(ssum, LAM * LAM))
                    scale = jnp.where(over, LAM * r, 1.0)
                    for j in range(D):
                        cj = jnp.full((16,), j, I32)
                        plsc.store_scatter(zbuf, [row, cj], zcols[j] * scale)
                    return gcarry

                lax.fori_loop(0, CH // 16, group, 0)

                if write_z:
                    pltpu.sync_copy(zbuf, z_out.at[pl.ds(eb, CH)])
                for j in range(NSUB // 2):
                    jj = h * (NSUB // 2) + j
                    pltpu.sync_copy(zbuf.at[pl.ds(j * SUB, SUB)],
                                    p_sh.at[idx_s.at[jj]], add=True)
                    pltpu.sync_copy(zbuf.at[pl.ds(j * SUB, SUB)],
                                    q_sh.at[idx_d.at[jj]], add=True)
            return carry

        lax.fori_loop(0, CHUNKS, chunk, 0)
        plsc.subcore_barrier()
        rb = s * RPT
        pltpu.sync_copy(p_sh.at[pl.ds(rb, RPT)], p_out.at[c, pl.ds(rb, RPT)])
        pltpu.sync_copy(q_sh.at[pl.ds(rb, RPT)], q_out.at[c, pl.ds(rb, RPT)])

    return pl.kernel(body, out_type=tuple(out_type), mesh=_mesh(),
                     scratch_types=scratch,
                     name="sc_dual",
                     compiler_params=_SC_PARAMS)


def _mlp(feat, W1, b1, W2, b2):
    def body(f_ref, w1_ref, b1_ref, w2_ref, b2_ref, o_ref):
        h1 = jnp.dot(f_ref[...], w1_ref[...], preferred_element_type=F32)
        h1 = jnp.maximum(h1 + b1_ref[...], 0.0)
        o_ref[...] = jnp.dot(h1, w2_ref[...],
                             preferred_element_type=F32) + b2_ref[...]

    return pl.pallas_call(
        body,
        grid=(10,),
        in_specs=[
            pl.BlockSpec((1000, 128), lambda i: (i, 0)),
            pl.BlockSpec((128, 64), lambda i: (0, 0)),
            pl.BlockSpec((1, 64), lambda i: (0, 0)),
            pl.BlockSpec((64, 32), lambda i: (0, 0)),
            pl.BlockSpec((1, 32), lambda i: (0, 0)),
        ],
        out_specs=pl.BlockSpec((1000, 32), lambda i: (i, 0)),
        out_shape=jax.ShapeDtypeStruct((N, D), F32),
    )(feat, W1, b1.reshape(1, 64), W2, b2.reshape(1, 32))


_NB = 2528


def _nspec(shape3=False, width=D):
    if shape3:
        return pl.BlockSpec((NC, _NB, width), lambda i: (0, i, 0))
    return pl.BlockSpec((_NB, width), lambda i: (i, 0))


def _nshape():
    return jax.ShapeDtypeStruct((NP, D), F32)


def _prep(dego, degi, h_pad):
    def body(do_ref, di_ref, h_ref, dob_ref, dib_ref, xs_ref):
        dso = do_ref[0, :, 0:1] + do_ref[1, :, 0:1]
        dsi = di_ref[0, :, 0:1] + di_ref[1, :, 0:1]
        dob = jnp.broadcast_to(lax.rsqrt(jnp.maximum(dso, 1.0)), (_NB, D))
        dib = jnp.broadcast_to(lax.rsqrt(jnp.maximum(dsi, 1.0)), (_NB, D))
        dob_ref[...] = dob
        dib_ref[...] = dib
        xs_ref[...] = dob * h_ref[...]

    return pl.pallas_call(
        body,
        grid=(NP // _NB,),
        in_specs=[_nspec(True, 16), _nspec(True, 16), _nspec()],
        out_specs=[_nspec(), _nspec(), _nspec()],
        out_shape=[_nshape(), _nshape(), _nshape()],
    )(dego, degi, h_pad)


def _node1(A, h_pad, dob, dib, P=None, Q=None):
    have_pq = P is not None

    def body(*refs):
        a_ref, h_ref, dob_ref, dib_ref = refs[:4]
        i = 4
        if have_pq:
            p_ref, q_ref = refs[i], refs[i + 1]
            i += 2
        y_ref, xbs_ref, xbd_ref = refs[i:]
        dob_v, dib_v = dob_ref[...], dib_ref[...]
        y = GAMMA * h_ref[...] + (1.0 - GAMMA) * dib_v * (
            a_ref[0] + a_ref[1])
        xbar = y
        if have_pq:
            u = dob_v * (p_ref[0] + p_ref[1]) - dib_v * (q_ref[0] + q_ref[1])
            xbar = y - GAMMA * u
        y_ref[...] = y
        xbs_ref[...] = dob_v * xbar
        xbd_ref[...] = dib_v * xbar

    in_specs = [_nspec(True), _nspec(), _nspec(), _nspec()]
    args = [A, h_pad, dob, dib]
    if have_pq:
        in_specs += [_nspec(True), _nspec(True)]
        args += [P, Q]
    return pl.pallas_call(
        body,
        grid=(NP // _NB,),
        in_specs=in_specs,
        out_specs=[_nspec(), _nspec(), _nspec()],
        out_shape=[_nshape(), _nshape(), _nshape()],
    )(*args)


def _node2(y, P, Q, dob, dib, want_xs):
    def body(*refs):
        y_ref, p_ref, q_ref, dob_ref, dib_ref = refs[:5]
        outs = refs[5:]
        dob_v, dib_v = dob_ref[...], dib_ref[...]
        u = dob_v * (p_ref[0] + p_ref[1]) - dib_v * (q_ref[0] + q_ref[1])
        x = y_ref[...] - GAMMA * u
        outs[0][...] = x
        if want_xs:
            outs[1][...] = dob_v * x

    n_out = 2 if want_xs else 1
    return pl.pallas_call(
        body,
        grid=(NP // _NB,),
        in_specs=[_nspec(), _nspec(True), _nspec(True), _nspec(), _nspec()],
        out_specs=[_nspec()] * n_out,
        out_shape=[_nshape()] * n_out,
    )(y, P, Q, dob, dib)


@jax.jit
def kernel(feat, edge_index, W1, b1, W2, b2):
    src = edge_index[0]
    dst = edge_index[1]
    pad = jnp.full((EP - E,), N, I32)
    src_p = jnp.concatenate([src, pad]).reshape(EP // SUB, SUB)
    dst_p = jnp.concatenate([dst, pad]).reshape(EP // SUB, SUB)

    h = _mlp(feat, W1, b1, W2, b2)
    h_pad = jnp.pad(h, ((0, NP - N), (0, 0)))

    dego, degi = _deg_kernel()(src_p, dst_p)
    dob, dib, xs = _prep(dego, degi, h_pad)

    pass_a = _pass_a_kernel()
    pass_b = _pass_b_kernel()
    z = jnp.zeros((EP, D), F32)

    A = pass_a(xs, src_p, dst_p)
    y, xbs, xbd = _node1(A, h_pad, dob, dib)
    z, P, Q = pass_b(xbs, xbd, src_p, dst_p, z)
    x, xs = _node2(y, P, Q, dob, dib, want_xs=True)

    A = pass_a(xs, src_p, dst_p)
    y, xbs, xbd = _node1(A, h_pad, dob, dib, P, Q)
    z, P, Q = pass_b(xbs, xbd, src_p, dst_p, z)
    x, xs = _node2(y, P, Q, dob, dib, want_xs=True)

    A = pass_a(xs, src_p, dst_p)
    y, xbs, xbd = _node1(A, h_pad, dob, dib, P, Q)
    z, P, Q = pass_b(xbs, xbd, src_p, dst_p, z)
    (x,) = _node2(y, P, Q, dob, dib, want_xs=False)

    return x[:N]


# speedup vs baseline: 6.5975x; 1.2779x over previous
import jax
import jax.numpy as jnp
from jax import lax
from jax.experimental import pallas as pl
from jax.experimental.pallas import tpu as pltpu
from jax.experimental.pallas import tpu_sc as plsc

N = 10000
NP = 10112
E = 320000
D = 32
NC = 2
NS = 16
NW = NC * NS
SUB = 128
EPT = 10240
EP = EPT * NW
IROWS = EPT // SUB
RPT = NP // NS
GAMMA = 0.25
BETA = 2.0
LAM = 3.0

F32 = jnp.float32
I32 = jnp.int32

_SC_PARAMS = pltpu.CompilerParams(use_tc_tiling_on_sc=False,
                                  needs_layout_passes=False)


def _mesh():
    return plsc.VectorSubcoreMesh(core_axis_name="c", subcore_axis_name="s",
                                  num_cores=NC, num_subcores=NS)


def _fill(ref, rows, width, value):
    v = jnp.full((16,), value, F32)

    def body(i, carry):
        for k in range(width // 16):
            ref[i, pl.ds(k * 16, 16)] = v
        return carry

    lax.fori_loop(0, rows, body, 0)


def _zero_acc(zsrc, acc_sh, s, ch):
    base = s * RPT
    off = 0
    while off < RPT:
        ln = min(ch, RPT - off)
        pltpu.sync_copy(zsrc.at[pl.ds(0, ln)], acc_sh.at[pl.ds(base + off, ln)])
        off += ln


def _nrsqrt(s):
    i = plsc.bitcast(s, I32)
    i = jnp.int32(0x5F3759DF) - lax.shift_right_logical(i, 1)
    y = plsc.bitcast(i, F32)
    for _ in range(3):
        y = y * (1.5 - 0.5 * s * y * y)
    return y


def _tile_ids():
    c = lax.axis_index("c")
    s = lax.axis_index("s")
    return c, s, c * NS + s


def _deg_kernel():
    NSUB = 8
    scratch = [
        pltpu.VMEM((NSUB, SUB), I32),
        pltpu.VMEM((NSUB, SUB), I32),
        pltpu.VMEM((SUB, 16), F32),
        pltpu.VMEM((RPT, 16), F32),
        pltpu.VMEM_SHARED((NP, 16), F32),
        pltpu.VMEM_SHARED((NP, 16), F32),
    ]
    out_type = (jax.ShapeDtypeStruct((NC, NP, 16), F32),
                jax.ShapeDtypeStruct((NC, NP, 16), F32))

    def body(src_h, dst_h, do_out, di_out, idx_s, idx_d, ones_b, zstage,
             do_sh, di_sh):
        c, s, wid = _tile_ids()
        _fill(ones_b, SUB, 16, 1.0)
        _fill(zstage, RPT, 16, 0.0)
        pltpu.sync_copy(zstage, do_sh.at[pl.ds(s * RPT, RPT)])
        pltpu.sync_copy(zstage, di_sh.at[pl.ds(s * RPT, RPT)])
        plsc.subcore_barrier()

        def chunk(ch, carry):
            rb = wid * IROWS + ch * NSUB
            pltpu.sync_copy(src_h.at[pl.ds(rb, NSUB)], idx_s)
            pltpu.sync_copy(dst_h.at[pl.ds(rb, NSUB)], idx_d)
            for j in range(NSUB):
                pltpu.sync_copy(ones_b, do_sh.at[idx_s.at[j]], add=True)
                pltpu.sync_copy(ones_b, di_sh.at[idx_d.at[j]], add=True)
            return carry

        lax.fori_loop(0, IROWS // NSUB, chunk, 0)
        plsc.subcore_barrier()
        rb = s * RPT
        pltpu.sync_copy(do_sh.at[pl.ds(rb, RPT)], do_out.at[c, pl.ds(rb, RPT)])
        pltpu.sync_copy(di_sh.at[pl.ds(rb, RPT)], di_out.at[c, pl.ds(rb, RPT)])

    return pl.kernel(body, out_type=out_type, mesh=_mesh(),
                     scratch_types=scratch, name="sc_degrees",
                     compiler_params=_SC_PARAMS)


def _pass_a_kernel():
    CH = 512
    TPS = CH // SUB
    H = EPT // CH
    scratch = [
        pltpu.VMEM((IROWS, SUB), I32),
        pltpu.VMEM((IROWS, SUB), I32),
        pltpu.VMEM((CH, D), F32),
        pltpu.VMEM((CH, D), F32),
        pltpu.VMEM_SHARED((NP, D), F32),
        pltpu.SemaphoreType.DMA,
        pltpu.SemaphoreType.DMA,
        pltpu.SemaphoreType.DMA,
        pltpu.SemaphoreType.DMA,
    ]
    out_type = jax.ShapeDtypeStruct((NC, NP, D), F32)

    def body(xs_h, src_h, dst_h, a_out, idx_s, idx_d, r0, r1, a_sh,
             g0, g1, s0, s1):
        ROWS = (r0, r1)
        GS = (g0, g1)
        SS = (s0, s1)
        c, s, wid = _tile_ids()
        pltpu.sync_copy(src_h.at[pl.ds(wid * IROWS, IROWS)], idx_s)
        pltpu.sync_copy(dst_h.at[pl.ds(wid * IROWS, IROWS)], idx_d)
        _fill(r0, CH, D, 0.0)
        _zero_acc(r0, a_sh, s, CH)
        plsc.subcore_barrier()

        def gathers(t, p, fire):
            for u in range(TPS):
                cp = pltpu.make_async_copy(
                    xs_h.at[idx_s.at[t * TPS + u]],
                    ROWS[p].at[pl.ds(u * SUB, SUB)], GS[p])
                cp.start() if fire else cp.wait()

        def scatters(t, p, fire):
            for u in range(TPS):
                cp = pltpu.make_async_copy(
                    ROWS[p].at[pl.ds(u * SUB, SUB)],
                    a_sh.at[idx_d.at[t * TPS + u]], SS[p])
                cp.start(add=True) if fire else cp.wait()

        gathers(0, 0, True)
        gathers(1, 1, True)

        def step(k, carry):
            t0 = 2 * k
            gathers(t0, 0, False)
            scatters(t0, 0, True)
            gathers(t0 + 1, 1, False)
            scatters(t0 + 1, 1, True)
            scatters(t0, 0, False)
            gathers(t0 + 2, 0, True)
            scatters(t0 + 1, 1, False)
            gathers(t0 + 3, 1, True)
            return carry

        lax.fori_loop(0, H // 2 - 1, step, 0)
        gathers(H - 2, 0, False)
        scatters(H - 2, 0, True)
        gathers(H - 1, 1, False)
        scatters(H - 1, 1, True)
        scatters(H - 2, 0, False)
        scatters(H - 1, 1, False)
        plsc.subcore_barrier()
        rb = s * RPT
        pltpu.sync_copy(a_sh.at[pl.ds(rb, RPT)], a_out.at[c, pl.ds(rb, RPT)])

    return pl.kernel(body, out_type=out_type, mesh=_mesh(),
                     scratch_types=scratch, name="sc_adj",
                     compiler_params=_SC_PARAMS)


def _pass_b_kernel():
    CH = 256
    TPS = CH // SUB
    H = EPT // CH
    scratch = [
        pltpu.VMEM((IROWS, SUB), I32),
        pltpu.VMEM((IROWS, SUB), I32),
        pltpu.VMEM((CH, D), F32),
        pltpu.VMEM((CH, D), F32),
        pltpu.VMEM((CH, D), F32),
        pltpu.VMEM((CH, D), F32),
        pltpu.VMEM((CH, D), F32),
        pltpu.VMEM((CH, D), F32),
        pltpu.VMEM_SHARED((NP, D), F32),
        pltpu.VMEM_SHARED((NP, D), F32),
        pltpu.SemaphoreType.DMA,
        pltpu.SemaphoreType.DMA,
        pltpu.SemaphoreType.DMA,
        pltpu.SemaphoreType.DMA,
        pltpu.SemaphoreType.DMA,
        pltpu.SemaphoreType.DMA,
        pltpu.SemaphoreType.DMA,
        pltpu.SemaphoreType.DMA,
    ]
    out_type = (jax.ShapeDtypeStruct((EP, D), F32),
                jax.ShapeDtypeStruct((NC, NP, D), F32),
                jax.ShapeDtypeStruct((NC, NP, D), F32))

    def body(xbs_h, xbd_h, src_h, dst_h, z_in, z_out, p_out, q_out,
             idx_s, idx_d, ra0, ra1, rb0, rb1, zb0, zb1, p_sh, q_sh,
             gi0, gi1, gz0, gz1, si0, si1, sz0, sz1):
        RA = (ra0, ra1)
        RB = (rb0, rb1)
        ZB = (zb0, zb1)
        GI = (gi0, gi1)
        GZ = (gz0, gz1)
        SI = (si0, si1)
        SZ = (sz0, sz1)
        c, s, wid = _tile_ids()
        pltpu.sync_copy(src_h.at[pl.ds(wid * IROWS, IROWS)], idx_s)
        pltpu.sync_copy(dst_h.at[pl.ds(wid * IROWS, IROWS)], idx_d)
        _fill(zb0, CH, D, 0.0)
        _zero_acc(zb0, p_sh, s, CH)
        _zero_acc(zb0, q_sh, s, CH)
        plsc.subcore_barrier()

        iota = lax.iota(I32, 16)
        ebase = wid * EPT

        def gathers(t, p, fire):
            cps = []
            for u in range(TPS):
                r = t * TPS + u
                cps.append(pltpu.make_async_copy(
                    xbs_h.at[idx_s.at[r]],
                    RA[p].at[pl.ds(u * SUB, SUB)], GI[p]))
                cps.append(pltpu.make_async_copy(
                    xbd_h.at[idx_d.at[r]],
                    RB[p].at[pl.ds(u * SUB, SUB)], GI[p]))
            cps.append(pltpu.make_async_copy(
                z_in.at[pl.ds(ebase + t * CH, CH)], ZB[p], GZ[p]))
            for cp in cps:
                cp.start() if fire else cp.wait()

        def scatters(t, p, fire):
            cp = pltpu.make_async_copy(
                ZB[p], z_out.at[pl.ds(ebase + t * CH, CH)], SZ[p])
            cp.start() if fire else cp.wait()
            for u in range(TPS):
                r = t * TPS + u
                cp = pltpu.make_async_copy(
                    ZB[p].at[pl.ds(u * SUB, SUB)], p_sh.at[idx_s.at[r]], SI[p])
                cp.start(add=True) if fire else cp.wait()
                cp = pltpu.make_async_copy(
                    ZB[p].at[pl.ds(u * SUB, SUB)], q_sh.at[idx_d.at[r]], SI[p])
                cp.start(add=True) if fire else cp.wait()

        def compute(p):
            def group(g, gcarry):
                row = g * 16 + iota
                ssum = jnp.zeros((16,), F32)
                zcols = []
                for j in range(D):
                    cj = jnp.full((16,), j, I32)
                    a = plsc.load_gather(RA[p], [row, cj])
                    b = plsc.load_gather(RB[p], [row, cj])
                    zb = BETA * (a - b) + plsc.load_gather(ZB[p], [row, cj])
                    ssum = ssum + zb * zb
                    zcols.append(zb)
                over = ssum > LAM * LAM
                r = _nrsqrt(jnp.maximum(ssum, LAM * LAM))
                scale = jnp.where(over, LAM * r, 1.0)
                for j in range(D):
                    cj = jnp.full((16,), j, I32)
                    plsc.store_scatter(ZB[p], [row, cj], zcols[j] * scale)
                return gcarry

            lax.fori_loop(0, CH // 16, group, 0)

        gathers(0, 0, True)
        gathers(1, 1, True)

        def step(k, carry):
            t0 = 2 * k
            gathers(t0, 0, False)
            compute(0)
            scatters(t0, 0, True)
            gathers(t0 + 1, 1, False)
            compute(1)
            scatters(t0 + 1, 1, True)
            scatters(t0, 0, False)
            gathers(t0 + 2, 0, True)
            scatters(t0 + 1, 1, False)
            gathers(t0 + 3, 1, True)
            return carry

        lax.fori_loop(0, H // 2 - 1, step, 0)
        gathers(H - 2, 0, False)
        compute(0)
        scatters(H - 2, 0, True)
        gathers(H - 1, 1, False)
        compute(1)
        scatters(H - 1, 1, True)
        scatters(H - 2, 0, False)
        scatters(H - 1, 1, False)
        plsc.subcore_barrier()
        rb = s * RPT
        pltpu.sync_copy(p_sh.at[pl.ds(rb, RPT)], p_out.at[c, pl.ds(rb, RPT)])
        pltpu.sync_copy(q_sh.at[pl.ds(rb, RPT)], q_out.at[c, pl.ds(rb, RPT)])

    return pl.kernel(body, out_type=out_type, mesh=_mesh(),
                     scratch_types=scratch, name="sc_dual",
                     compiler_params=_SC_PARAMS)


def _mlp(feat, W1, b1, W2, b2):
    def body(f_ref, w1_ref, b1_ref, w2_ref, b2_ref, o_ref):
        h1 = jnp.dot(f_ref[...], w1_ref[...], preferred_element_type=F32)
        h1 = jnp.maximum(h1 + b1_ref[...], 0.0)
        o_ref[...] = jnp.dot(h1, w2_ref[...],
                             preferred_element_type=F32) + b2_ref[...]

    return pl.pallas_call(
        body,
        grid=(10,),
        in_specs=[
            pl.BlockSpec((1000, 128), lambda i: (i, 0)),
            pl.BlockSpec((128, 64), lambda i: (0, 0)),
            pl.BlockSpec((1, 64), lambda i: (0, 0)),
            pl.BlockSpec((64, 32), lambda i: (0, 0)),
            pl.BlockSpec((1, 32), lambda i: (0, 0)),
        ],
        out_specs=pl.BlockSpec((1000, 32), lambda i: (i, 0)),
        out_shape=jax.ShapeDtypeStruct((N, D), F32),
    )(feat, W1, b1.reshape(1, 64), W2, b2.reshape(1, 32))


_NB = 2528


def _nspec(shape3=False, width=D):
    if shape3:
        return pl.BlockSpec((NC, _NB, width), lambda i: (0, i, 0))
    return pl.BlockSpec((_NB, width), lambda i: (i, 0))


def _nshape():
    return jax.ShapeDtypeStruct((NP, D), F32)


def _prep(dego, degi, h_pad):
    def body(do_ref, di_ref, h_ref, dob_ref, dib_ref, xs_ref):
        dso = do_ref[0, :, 0:1] + do_ref[1, :, 0:1]
        dsi = di_ref[0, :, 0:1] + di_ref[1, :, 0:1]
        dob = jnp.broadcast_to(lax.rsqrt(jnp.maximum(dso, 1.0)), (_NB, D))
        dib = jnp.broadcast_to(lax.rsqrt(jnp.maximum(dsi, 1.0)), (_NB, D))
        dob_ref[...] = dob
        dib_ref[...] = dib
        xs_ref[...] = dob * h_ref[...]

    return pl.pallas_call(
        body,
        grid=(NP // _NB,),
        in_specs=[_nspec(True, 16), _nspec(True, 16), _nspec()],
        out_specs=[_nspec(), _nspec(), _nspec()],
        out_shape=[_nshape(), _nshape(), _nshape()],
    )(dego, degi, h_pad)


def _node1(A, h_pad, dob, dib, P=None, Q=None):
    have_pq = P is not None

    def body(*refs):
        a_ref, h_ref, dob_ref, dib_ref = refs[:4]
        i = 4
        if have_pq:
            p_ref, q_ref = refs[i], refs[i + 1]
            i += 2
        y_ref, xbs_ref, xbd_ref = refs[i:]
        dob_v, dib_v = dob_ref[...], dib_ref[...]
        y = GAMMA * h_ref[...] + (1.0 - GAMMA) * dib_v * (
            a_ref[0] + a_ref[1])
        xbar = y
        if have_pq:
            u = dob_v * (p_ref[0] + p_ref[1]) - dib_v * (q_ref[0] + q_ref[1])
            xbar = y - GAMMA * u
        y_ref[...] = y
        xbs_ref[...] = dob_v * xbar
        xbd_ref[...] = dib_v * xbar

    in_specs = [_nspec(True), _nspec(), _nspec(), _nspec()]
    args = [A, h_pad, dob, dib]
    if have_pq:
        in_specs += [_nspec(True), _nspec(True)]
        args += [P, Q]
    return pl.pallas_call(
        body,
        grid=(NP // _NB,),
        in_specs=in_specs,
        out_specs=[_nspec(), _nspec(), _nspec()],
        out_shape=[_nshape(), _nshape(), _nshape()],
    )(*args)


def _node2(y, P, Q, dob, dib, want_xs):
    def body(*refs):
        y_ref, p_ref, q_ref, dob_ref, dib_ref = refs[:5]
        outs = refs[5:]
        dob_v, dib_v = dob_ref[...], dib_ref[...]
        u = dob_v * (p_ref[0] + p_ref[1]) - dib_v * (q_ref[0] + q_ref[1])
        x = y_ref[...] - GAMMA * u
        outs[0][...] = x
        if want_xs:
            outs[1][...] = dob_v * x

    n_out = 2 if want_xs else 1
    return pl.pallas_call(
        body,
        grid=(NP // _NB,),
        in_specs=[_nspec(), _nspec(True), _nspec(True), _nspec(), _nspec()],
        out_specs=[_nspec()] * n_out,
        out_shape=[_nshape()] * n_out,
    )(y, P, Q, dob, dib)


@jax.jit
def kernel(feat, edge_index, W1, b1, W2, b2):
    src = edge_index[0]
    dst = edge_index[1]
    pad = jnp.full((EP - E,), N, I32)
    src_p = jnp.concatenate([src, pad]).reshape(EP // SUB, SUB)
    dst_p = jnp.concatenate([dst, pad]).reshape(EP // SUB, SUB)

    h = _mlp(feat, W1, b1, W2, b2)
    h_pad = jnp.pad(h, ((0, NP - N), (0, 0)))

    dego, degi = _deg_kernel()(src_p, dst_p)
    dob, dib, xs = _prep(dego, degi, h_pad)

    pass_a = _pass_a_kernel()
    pass_b = _pass_b_kernel()
    z = jnp.zeros((EP, D), F32)

    A = pass_a(xs, src_p, dst_p)
    y, xbs, xbd = _node1(A, h_pad, dob, dib)
    z, P, Q = pass_b(xbs, xbd, src_p, dst_p, z)
    x, xs = _node2(y, P, Q, dob, dib, want_xs=True)

    A = pass_a(xs, src_p, dst_p)
    y, xbs, xbd = _node1(A, h_pad, dob, dib, P, Q)
    z, P, Q = pass_b(xbs, xbd, src_p, dst_p, z)
    x, xs = _node2(y, P, Q, dob, dib, want_xs=True)

    A = pass_a(xs, src_p, dst_p)
    y, xbs, xbd = _node1(A, h_pad, dob, dib, P, Q)
    z, P, Q = pass_b(xbs, xbd, src_p, dst_p, z)
    (x,) = _node2(y, P, Q, dob, dib, want_xs=False)

    return x[:N]


# speedup vs baseline: 13.1834x; 1.9982x over previous
import jax
import jax.numpy as jnp
from jax import lax
from jax.experimental import pallas as pl
from jax.experimental.pallas import tpu as pltpu
from jax.experimental.pallas import tpu_sc as plsc

N = 10000
NP = 10112
E = 320000
D = 32
NC = 2
NS = 16
NW = NC * NS
SUB = 128
EPT = 10240
EP = EPT * NW
IROWS = EPT // SUB
RPT = NP // NS
GAMMA = 0.25
BETA = 2.0
LAM = 3.0

F32 = jnp.float32
I32 = jnp.int32

_SC_PARAMS = pltpu.CompilerParams(use_tc_tiling_on_sc=False,
                                  needs_layout_passes=False)


def _mesh():
    return plsc.VectorSubcoreMesh(core_axis_name="c", subcore_axis_name="s",
                                  num_cores=NC, num_subcores=NS)


def _fill(ref, rows, width, value):
    v = jnp.full((16,), value, F32)

    def body(i, carry):
        for k in range(width // 16):
            ref[i, pl.ds(k * 16, 16)] = v
        return carry

    lax.fori_loop(0, rows, body, 0)


def _zero_acc(zsrc, acc_sh, s, ch):
    base = s * RPT
    off = 0
    while off < RPT:
        ln = min(ch, RPT - off)
        pltpu.sync_copy(zsrc.at[pl.ds(0, ln)], acc_sh.at[pl.ds(base + off, ln)])
        off += ln


def _nrsqrt(s):
    i = plsc.bitcast(s, I32)
    i = jnp.int32(0x5F3759DF) - lax.shift_right_logical(i, 1)
    y = plsc.bitcast(i, F32)
    for _ in range(3):
        y = y * (1.5 - 0.5 * s * y * y)
    return y


def _nrsqrt_scalar(s):
    i = lax.bitcast_convert_type(s, I32)
    i = jnp.int32(0x5F3759DF) - lax.shift_right_logical(i, 1)
    y = lax.bitcast_convert_type(i, F32)
    for _ in range(3):
        y = y * (1.5 - 0.5 * s * y * y)
    return y


def _tile_ids():
    c = lax.axis_index("c")
    s = lax.axis_index("s")
    return c, s, c * NS + s


def _deg_kernel():
    NSUB = 8
    scratch = [
        pltpu.VMEM((NSUB, SUB), I32),
        pltpu.VMEM((NSUB, SUB), I32),
        pltpu.VMEM((SUB, 16), F32),
        pltpu.VMEM((RPT, 16), F32),
        pltpu.VMEM_SHARED((NP, 16), F32),
        pltpu.VMEM_SHARED((NP, 16), F32),
    ]
    out_type = (jax.ShapeDtypeStruct((NC, NP, 16), F32),
                jax.ShapeDtypeStruct((NC, NP, 16), F32))

    def body(src_h, dst_h, do_out, di_out, idx_s, idx_d, ones_b, zstage,
             do_sh, di_sh):
        c, s, wid = _tile_ids()
        _fill(ones_b, SUB, 16, 1.0)
        _fill(zstage, RPT, 16, 0.0)
        pltpu.sync_copy(zstage, do_sh.at[pl.ds(s * RPT, RPT)])
        pltpu.sync_copy(zstage, di_sh.at[pl.ds(s * RPT, RPT)])
        plsc.subcore_barrier()

        def chunk(ch, carry):
            rb = wid * IROWS + ch * NSUB
            pltpu.sync_copy(src_h.at[pl.ds(rb, NSUB)], idx_s)
            pltpu.sync_copy(dst_h.at[pl.ds(rb, NSUB)], idx_d)
            for j in range(NSUB):
                pltpu.sync_copy(ones_b, do_sh.at[idx_s.at[j]], add=True)
                pltpu.sync_copy(ones_b, di_sh.at[idx_d.at[j]], add=True)
            return carry

        lax.fori_loop(0, IROWS // NSUB, chunk, 0)
        plsc.subcore_barrier()
        rb = s * RPT
        pltpu.sync_copy(do_sh.at[pl.ds(rb, RPT)], do_out.at[c, pl.ds(rb, RPT)])
        pltpu.sync_copy(di_sh.at[pl.ds(rb, RPT)], di_out.at[c, pl.ds(rb, RPT)])

    return pl.kernel(body, out_type=out_type, mesh=_mesh(),
                     scratch_types=scratch, name="sc_degrees",
                     compiler_params=_SC_PARAMS)


def _pass_a_kernel():
    CH = 512
    TPS = CH // SUB
    H = EPT // CH
    scratch = [
        pltpu.VMEM((IROWS, SUB), I32),
        pltpu.VMEM((IROWS, SUB), I32),
        pltpu.VMEM((CH, D), F32),
        pltpu.VMEM((CH, D), F32),
        pltpu.VMEM_SHARED((NP, D), F32),
        pltpu.SemaphoreType.DMA,
        pltpu.SemaphoreType.DMA,
        pltpu.SemaphoreType.DMA,
        pltpu.SemaphoreType.DMA,
    ]
    out_type = jax.ShapeDtypeStruct((NC, NP, D), F32)

    def body(xs_h, src_h, dst_h, a_out, idx_s, idx_d, r0, r1, a_sh,
             g0, g1, s0, s1):
        ROWS = (r0, r1)
        GS = (g0, g1)
        SS = (s0, s1)
        c, s, wid = _tile_ids()
        pltpu.sync_copy(src_h.at[pl.ds(wid * IROWS, IROWS)], idx_s)
        pltpu.sync_copy(dst_h.at[pl.ds(wid * IROWS, IROWS)], idx_d)
        _fill(r0, CH, D, 0.0)
        _zero_acc(r0, a_sh, s, CH)
        plsc.subcore_barrier()

        def gathers(t, p, fire):
            for u in range(TPS):
                cp = pltpu.make_async_copy(
                    xs_h.at[idx_s.at[t * TPS + u]],
                    ROWS[p].at[pl.ds(u * SUB, SUB)], GS[p])
                cp.start() if fire else cp.wait()

        def scatters(t, p, fire):
            for u in range(TPS):
                cp = pltpu.make_async_copy(
                    ROWS[p].at[pl.ds(u * SUB, SUB)],
                    a_sh.at[idx_d.at[t * TPS + u]], SS[p])
                cp.start(add=True) if fire else cp.wait()

        gathers(0, 0, True)
        gathers(1, 1, True)

        def step(k, carry):
            t0 = 2 * k
            gathers(t0, 0, False)
            scatters(t0, 0, True)
            gathers(t0 + 1, 1, False)
            scatters(t0 + 1, 1, True)
            scatters(t0, 0, False)
            gathers(t0 + 2, 0, True)
            scatters(t0 + 1, 1, False)
            gathers(t0 + 3, 1, True)
            return carry

        lax.fori_loop(0, H // 2 - 1, step, 0)
        gathers(H - 2, 0, False)
        scatters(H - 2, 0, True)
        gathers(H - 1, 1, False)
        scatters(H - 1, 1, True)
        scatters(H - 2, 0, False)
        scatters(H - 1, 1, False)
        plsc.subcore_barrier()
        rb = s * RPT
        pltpu.sync_copy(a_sh.at[pl.ds(rb, RPT)], a_out.at[c, pl.ds(rb, RPT)])

    return pl.kernel(body, out_type=out_type, mesh=_mesh(),
                     scratch_types=scratch, name="sc_adj",
                     compiler_params=_SC_PARAMS)


def _pass_b_kernel():
    CH = 256
    TPS = CH // SUB
    H = EPT // CH
    scratch = [
        pltpu.VMEM((IROWS, SUB), I32),
        pltpu.VMEM((IROWS, SUB), I32),
        pltpu.VMEM((CH, D), F32),
        pltpu.VMEM((CH, D), F32),
        pltpu.VMEM((CH, D), F32),
        pltpu.VMEM((CH, D), F32),
        pltpu.VMEM((CH, D), F32),
        pltpu.VMEM((CH, D), F32),
        pltpu.VMEM_SHARED((NP, D), F32),
        pltpu.VMEM_SHARED((NP, D), F32),
        pltpu.SemaphoreType.DMA,
        pltpu.SemaphoreType.DMA,
        pltpu.SemaphoreType.DMA,
        pltpu.SemaphoreType.DMA,
        pltpu.SemaphoreType.DMA,
        pltpu.SemaphoreType.DMA,
        pltpu.SemaphoreType.DMA,
        pltpu.SemaphoreType.DMA,
    ]
    out_type = (jax.ShapeDtypeStruct((EP, D), F32),
                jax.ShapeDtypeStruct((NC, NP, D), F32),
                jax.ShapeDtypeStruct((NC, NP, D), F32))

    def body(xbs_h, xbd_h, src_h, dst_h, z_in, z_out, p_out, q_out,
             idx_s, idx_d, ra0, ra1, rb0, rb1, zb0, zb1, p_sh, q_sh,
             gi0, gi1, gz0, gz1, si0, si1, sz0, sz1):
        RA = (ra0, ra1)
        RB = (rb0, rb1)
        ZB = (zb0, zb1)
        GI = (gi0, gi1)
        GZ = (gz0, gz1)
        SI = (si0, si1)
        SZ = (sz0, sz1)
        c, s, wid = _tile_ids()
        pltpu.sync_copy(src_h.at[pl.ds(wid * IROWS, IROWS)], idx_s)
        pltpu.sync_copy(dst_h.at[pl.ds(wid * IROWS, IROWS)], idx_d)
        _fill(zb0, CH, D, 0.0)
        _zero_acc(zb0, p_sh, s, CH)
        _zero_acc(zb0, q_sh, s, CH)
        plsc.subcore_barrier()

        iota = lax.iota(I32, 16)
        ebase = wid * EPT

        def gathers(t, p, fire):
            cps = []
            for u in range(TPS):
                r = t * TPS + u
                cps.append(pltpu.make_async_copy(
                    xbs_h.at[idx_s.at[r]],
                    RA[p].at[pl.ds(u * SUB, SUB)], GI[p]))
                cps.append(pltpu.make_async_copy(
                    xbd_h.at[idx_d.at[r]],
                    RB[p].at[pl.ds(u * SUB, SUB)], GI[p]))
            cps.append(pltpu.make_async_copy(
                z_in.at[pl.ds(ebase + t * CH, CH)], ZB[p], GZ[p]))
            for cp in cps:
                cp.start() if fire else cp.wait()

        def scatters(t, p, fire):
            cp = pltpu.make_async_copy(
                ZB[p], z_out.at[pl.ds(ebase + t * CH, CH)], SZ[p])
            cp.start() if fire else cp.wait()
            for u in range(TPS):
                r = t * TPS + u
                cp = pltpu.make_async_copy(
                    ZB[p].at[pl.ds(u * SUB, SUB)], p_sh.at[idx_s.at[r]], SI[p])
                cp.start(add=True) if fire else cp.wait()
                cp = pltpu.make_async_copy(
                    ZB[p].at[pl.ds(u * SUB, SUB)], q_sh.at[idx_d.at[r]], SI[p])
                cp.start(add=True) if fire else cp.wait()

        def compute(p):
            UNROLL = 8

            def group(g, gcarry):
                base = g * UNROLL
                for rr in range(UNROLL):
                    row = base + rr
                    halves = []
                    ssum = None
                    for hh in range(2):
                        sl = pl.ds(hh * 16, 16)
                        a = RA[p][row, sl]
                        b = RB[p][row, sl]
                        zb = BETA * (a - b) + ZB[p][row, sl]
                        sq = zb * zb
                        ssum = sq if ssum is None else ssum + sq
                        halves.append(zb)
                    sv = lax.reduce_sum_p.bind(ssum, axes=(0,))
                    over = sv > LAM * LAM
                    r = _nrsqrt_scalar(jnp.maximum(sv, LAM * LAM))
                    scale = jnp.where(over, LAM * r, 1.0)
                    for hh in range(2):
                        ZB[p][row, pl.ds(hh * 16, 16)] = halves[hh] * scale
                return gcarry

            lax.fori_loop(0, CH // UNROLL, group, 0)

        gathers(0, 0, True)
        gathers(1, 1, True)

        def step(k, carry):
            t0 = 2 * k
            gathers(t0, 0, False)
            compute(0)
            scatters(t0, 0, True)
            gathers(t0 + 1, 1, False)
            compute(1)
            scatters(t0 + 1, 1, True)
            scatters(t0, 0, False)
            gathers(t0 + 2, 0, True)
            scatters(t0 + 1, 1, False)
            gathers(t0 + 3, 1, True)
            return carry

        lax.fori_loop(0, H // 2 - 1, step, 0)
        gathers(H - 2, 0, False)
        compute(0)
        scatters(H - 2, 0, True)
        gathers(H - 1, 1, False)
        compute(1)
        scatters(H - 1, 1, True)
        scatters(H - 2, 0, False)
        scatters(H - 1, 1, False)
        plsc.subcore_barrier()
        rb = s * RPT
        pltpu.sync_copy(p_sh.at[pl.ds(rb, RPT)], p_out.at[c, pl.ds(rb, RPT)])
        pltpu.sync_copy(q_sh.at[pl.ds(rb, RPT)], q_out.at[c, pl.ds(rb, RPT)])

    return pl.kernel(body, out_type=out_type, mesh=_mesh(),
                     scratch_types=scratch, name="sc_dual",
                     compiler_params=_SC_PARAMS)


def _mlp(feat, W1, b1, W2, b2):
    def body(f_ref, w1_ref, b1_ref, w2_ref, b2_ref, o_ref):
        h1 = jnp.dot(f_ref[...], w1_ref[...], preferred_element_type=F32)
        h1 = jnp.maximum(h1 + b1_ref[...], 0.0)
        o_ref[...] = jnp.dot(h1, w2_ref[...],
                             preferred_element_type=F32) + b2_ref[...]

    return pl.pallas_call(
        body,
        grid=(10,),
        in_specs=[
            pl.BlockSpec((1000, 128), lambda i: (i, 0)),
            pl.BlockSpec((128, 64), lambda i: (0, 0)),
            pl.BlockSpec((1, 64), lambda i: (0, 0)),
            pl.BlockSpec((64, 32), lambda i: (0, 0)),
            pl.BlockSpec((1, 32), lambda i: (0, 0)),
        ],
        out_specs=pl.BlockSpec((1000, 32), lambda i: (i, 0)),
        out_shape=jax.ShapeDtypeStruct((N, D), F32),
    )(feat, W1, b1.reshape(1, 64), W2, b2.reshape(1, 32))


_NB = 2528


def _nspec(shape3=False, width=D):
    if shape3:
        return pl.BlockSpec((NC, _NB, width), lambda i: (0, i, 0))
    return pl.BlockSpec((_NB, width), lambda i: (i, 0))


def _nshape():
    return jax.ShapeDtypeStruct((NP, D), F32)


def _prep(dego, degi, h_pad):
    def body(do_ref, di_ref, h_ref, dob_ref, dib_ref, xs_ref):
        dso = do_ref[0, :, 0:1] + do_ref[1, :, 0:1]
        dsi = di_ref[0, :, 0:1] + di_ref[1, :, 0:1]
        dob = jnp.broadcast_to(lax.rsqrt(jnp.maximum(dso, 1.0)), (_NB, D))
        dib = jnp.broadcast_to(lax.rsqrt(jnp.maximum(dsi, 1.0)), (_NB, D))
        dob_ref[...] = dob
        dib_ref[...] = dib
        xs_ref[...] = dob * h_ref[...]

    return pl.pallas_call(
        body,
        grid=(NP // _NB,),
        in_specs=[_nspec(True, 16), _nspec(True, 16), _nspec()],
        out_specs=[_nspec(), _nspec(), _nspec()],
        out_shape=[_nshape(), _nshape(), _nshape()],
    )(dego, degi, h_pad)


def _node1(A, h_pad, dob, dib, P=None, Q=None):
    have_pq = P is not None

    def body(*refs):
        a_ref, h_ref, dob_ref, dib_ref = refs[:4]
        i = 4
        if have_pq:
            p_ref, q_ref = refs[i], refs[i + 1]
            i += 2
        y_ref, xbs_ref, xbd_ref = refs[i:]
        dob_v, dib_v = dob_ref[...], dib_ref[...]
        y = GAMMA * h_ref[...] + (1.0 - GAMMA) * dib_v * (
            a_ref[0] + a_ref[1])
        xbar = y
        if have_pq:
            u = dob_v * (p_ref[0] + p_ref[1]) - dib_v * (q_ref[0] + q_ref[1])
            xbar = y - GAMMA * u
        y_ref[...] = y
        xbs_ref[...] = dob_v * xbar
        xbd_ref[...] = dib_v * xbar

    in_specs = [_nspec(True), _nspec(), _nspec(), _nspec()]
    args = [A, h_pad, dob, dib]
    if have_pq:
        in_specs += [_nspec(True), _nspec(True)]
        args += [P, Q]
    return pl.pallas_call(
        body,
        grid=(NP // _NB,),
        in_specs=in_specs,
        out_specs=[_nspec(), _nspec(), _nspec()],
        out_shape=[_nshape(), _nshape(), _nshape()],
    )(*args)


def _node2(y, P, Q, dob, dib, want_xs):
    def body(*refs):
        y_ref, p_ref, q_ref, dob_ref, dib_ref = refs[:5]
        outs = refs[5:]
        dob_v, dib_v = dob_ref[...], dib_ref[...]
        u = dob_v * (p_ref[0] + p_ref[1]) - dib_v * (q_ref[0] + q_ref[1])
        x = y_ref[...] - GAMMA * u
        outs[0][...] = x
        if want_xs:
            outs[1][...] = dob_v * x

    n_out = 2 if want_xs else 1
    return pl.pallas_call(
        body,
        grid=(NP // _NB,),
        in_specs=[_nspec(), _nspec(True), _nspec(True), _nspec(), _nspec()],
        out_specs=[_nspec()] * n_out,
        out_shape=[_nshape()] * n_out,
    )(y, P, Q, dob, dib)


@jax.jit
def kernel(feat, edge_index, W1, b1, W2, b2):
    src = edge_index[0]
    dst = edge_index[1]
    pad = jnp.full((EP - E,), N, I32)
    src_p = jnp.concatenate([src, pad]).reshape(EP // SUB, SUB)
    dst_p = jnp.concatenate([dst, pad]).reshape(EP // SUB, SUB)

    h = _mlp(feat, W1, b1, W2, b2)
    h_pad = jnp.pad(h, ((0, NP - N), (0, 0)))

    dego, degi = _deg_kernel()(src_p, dst_p)
    dob, dib, xs = _prep(dego, degi, h_pad)

    pass_a = _pass_a_kernel()
    pass_b = _pass_b_kernel()
    z = jnp.zeros((EP, D), F32)

    A = pass_a(xs, src_p, dst_p)
    y, xbs, xbd = _node1(A, h_pad, dob, dib)
    z, P, Q = pass_b(xbs, xbd, src_p, dst_p, z)
    x, xs = _node2(y, P, Q, dob, dib, want_xs=True)

    A = pass_a(xs, src_p, dst_p)
    y, xbs, xbd = _node1(A, h_pad, dob, dib, P, Q)
    z, P, Q = pass_b(xbs, xbd, src_p, dst_p, z)
    x, xs = _node2(y, P, Q, dob, dib, want_xs=True)

    A = pass_a(xs, src_p, dst_p)
    y, xbs, xbd = _node1(A, h_pad, dob, dib, P, Q)
    z, P, Q = pass_b(xbs, xbd, src_p, dst_p, z)
    (x,) = _node2(y, P, Q, dob, dib, want_xs=False)

    return x[:N]


# speedup vs baseline: 13.6254x; 1.0335x over previous
import jax
import jax.numpy as jnp
from jax import lax
from jax.experimental import pallas as pl
from jax.experimental.pallas import tpu as pltpu
from jax.experimental.pallas import tpu_sc as plsc

N = 10000
NP = 10112
E = 320000
D = 32
NC = 2
NS = 16
NW = NC * NS
SUB = 128
EPT = 10240
EP = EPT * NW
IROWS = EPT // SUB
RPT = NP // NS
GAMMA = 0.25
BETA = 2.0
LAM = 3.0

F32 = jnp.float32
I32 = jnp.int32

_SC_PARAMS = pltpu.CompilerParams(use_tc_tiling_on_sc=False,
                                  needs_layout_passes=False)


def _mesh():
    return plsc.VectorSubcoreMesh(core_axis_name="c", subcore_axis_name="s",
                                  num_cores=NC, num_subcores=NS)


def _fill(ref, rows, width, value):
    v = jnp.full((16,), value, F32)

    def body(i, carry):
        for k in range(width // 16):
            ref[i, pl.ds(k * 16, 16)] = v
        return carry

    lax.fori_loop(0, rows, body, 0)


def _zero_acc(zsrc, acc_sh, s, ch):
    base = s * RPT
    off = 0
    while off < RPT:
        ln = min(ch, RPT - off)
        pltpu.sync_copy(zsrc.at[pl.ds(0, ln)], acc_sh.at[pl.ds(base + off, ln)])
        off += ln


def _nrsqrt(s):
    i = plsc.bitcast(s, I32)
    i = jnp.int32(0x5F3759DF) - lax.shift_right_logical(i, 1)
    y = plsc.bitcast(i, F32)
    for _ in range(3):
        y = y * (1.5 - 0.5 * s * y * y)
    return y


def _nrsqrt_scalar(s):
    i = lax.bitcast_convert_type(s, I32)
    i = jnp.int32(0x5F3759DF) - lax.shift_right_logical(i, 1)
    y = lax.bitcast_convert_type(i, F32)
    for _ in range(3):
        y = y * (1.5 - 0.5 * s * y * y)
    return y


def _tile_ids():
    c = lax.axis_index("c")
    s = lax.axis_index("s")
    return c, s, c * NS + s


def _deg_kernel():
    NSUB = 8
    scratch = [
        pltpu.VMEM((NSUB, SUB), I32),
        pltpu.VMEM((NSUB, SUB), I32),
        pltpu.VMEM((SUB, 16), F32),
        pltpu.VMEM((RPT, 16), F32),
        pltpu.VMEM_SHARED((NP, 16), F32),
        pltpu.VMEM_SHARED((NP, 16), F32),
    ]
    out_type = (jax.ShapeDtypeStruct((NC, NP, 16), F32),
                jax.ShapeDtypeStruct((NC, NP, 16), F32))

    def body(src_h, dst_h, do_out, di_out, idx_s, idx_d, ones_b, zstage,
             do_sh, di_sh):
        c, s, wid = _tile_ids()
        _fill(ones_b, SUB, 16, 1.0)
        _fill(zstage, RPT, 16, 0.0)
        pltpu.sync_copy(zstage, do_sh.at[pl.ds(s * RPT, RPT)])
        pltpu.sync_copy(zstage, di_sh.at[pl.ds(s * RPT, RPT)])
        plsc.subcore_barrier()

        def chunk(ch, carry):
            rb = wid * IROWS + ch * NSUB
            pltpu.sync_copy(src_h.at[pl.ds(rb, NSUB)], idx_s)
            pltpu.sync_copy(dst_h.at[pl.ds(rb, NSUB)], idx_d)
            for j in range(NSUB):
                pltpu.sync_copy(ones_b, do_sh.at[idx_s.at[j]], add=True)
                pltpu.sync_copy(ones_b, di_sh.at[idx_d.at[j]], add=True)
            return carry

        lax.fori_loop(0, IROWS // NSUB, chunk, 0)
        plsc.subcore_barrier()
        rb = s * RPT
        pltpu.sync_copy(do_sh.at[pl.ds(rb, RPT)], do_out.at[c, pl.ds(rb, RPT)])
        pltpu.sync_copy(di_sh.at[pl.ds(rb, RPT)], di_out.at[c, pl.ds(rb, RPT)])

    return pl.kernel(body, out_type=out_type, mesh=_mesh(),
                     scratch_types=scratch, name="sc_degrees",
                     compiler_params=_SC_PARAMS)


def _pass_a_kernel():
    CH = 1024
    TPS = CH // SUB
    H = EPT // CH
    scratch = [
        pltpu.VMEM((IROWS, SUB), I32),
        pltpu.VMEM((IROWS, SUB), I32),
        pltpu.VMEM((CH, D), F32),
        pltpu.VMEM((CH, D), F32),
        pltpu.VMEM_SHARED((NP, D), F32),
        pltpu.SemaphoreType.DMA,
        pltpu.SemaphoreType.DMA,
        pltpu.SemaphoreType.DMA,
        pltpu.SemaphoreType.DMA,
    ]
    out_type = jax.ShapeDtypeStruct((NC, NP, D), F32)

    def body(xs_h, src_h, dst_h, a_out, idx_s, idx_d, r0, r1, a_sh,
             g0, g1, s0, s1):
        ROWS = (r0, r1)
        GS = (g0, g1)
        SS = (s0, s1)
        c, s, wid = _tile_ids()
        pltpu.sync_copy(src_h.at[pl.ds(wid * IROWS, IROWS)], idx_s)
        pltpu.sync_copy(dst_h.at[pl.ds(wid * IROWS, IROWS)], idx_d)
        _fill(r0, CH, D, 0.0)
        _zero_acc(r0, a_sh, s, CH)
        plsc.subcore_barrier()

        def gathers(t, p, fire):
            for u in range(TPS):
                cp = pltpu.make_async_copy(
                    xs_h.at[idx_s.at[t * TPS + u]],
                    ROWS[p].at[pl.ds(u * SUB, SUB)], GS[p])
                cp.start() if fire else cp.wait()

        def scatters(t, p, fire):
            for u in range(TPS):
                cp = pltpu.make_async_copy(
                    ROWS[p].at[pl.ds(u * SUB, SUB)],
                    a_sh.at[idx_d.at[t * TPS + u]], SS[p])
                cp.start(add=True) if fire else cp.wait()

        gathers(0, 0, True)
        gathers(1, 1, True)

        def step(k, carry):
            t0 = 2 * k
            gathers(t0, 0, False)
            scatters(t0, 0, True)
            gathers(t0 + 1, 1, False)
            scatters(t0 + 1, 1, True)
            scatters(t0, 0, False)
            gathers(t0 + 2, 0, True)
            scatters(t0 + 1, 1, False)
            gathers(t0 + 3, 1, True)
            return carry

        lax.fori_loop(0, H // 2 - 1, step, 0)
        gathers(H - 2, 0, False)
        scatters(H - 2, 0, True)
        gathers(H - 1, 1, False)
        scatters(H - 1, 1, True)
        scatters(H - 2, 0, False)
        scatters(H - 1, 1, False)
        plsc.subcore_barrier()
        rb = s * RPT
        pltpu.sync_copy(a_sh.at[pl.ds(rb, RPT)], a_out.at[c, pl.ds(rb, RPT)])

    return pl.kernel(body, out_type=out_type, mesh=_mesh(),
                     scratch_types=scratch, name="sc_adj",
                     compiler_params=_SC_PARAMS)


def _pass_b_kernel(read_z, write_z):
    CH = 256
    TPS = CH // SUB
    H = EPT // CH
    scratch = [
        pltpu.VMEM((IROWS, SUB), I32),
        pltpu.VMEM((IROWS, SUB), I32),
        pltpu.VMEM((CH, D), F32),
        pltpu.VMEM((CH, D), F32),
        pltpu.VMEM((CH, D), F32),
        pltpu.VMEM((CH, D), F32),
        pltpu.VMEM((CH, D), F32),
        pltpu.VMEM((CH, D), F32),
        pltpu.VMEM_SHARED((NP, D), F32),
        pltpu.VMEM_SHARED((NP, D), F32),
        pltpu.SemaphoreType.DMA,
        pltpu.SemaphoreType.DMA,
        pltpu.SemaphoreType.DMA,
        pltpu.SemaphoreType.DMA,
        pltpu.SemaphoreType.DMA,
        pltpu.SemaphoreType.DMA,
        pltpu.SemaphoreType.DMA,
        pltpu.SemaphoreType.DMA,
    ]
    out_type = ([jax.ShapeDtypeStruct((EP, D), F32)] if write_z else []) + [
        jax.ShapeDtypeStruct((NC, NP, D), F32),
        jax.ShapeDtypeStruct((NC, NP, D), F32)]
    out_type = tuple(out_type)

    def body(*refs):
        xbs_h, xbd_h, src_h, dst_h = refs[:4]
        i = 4
        z_in = refs[i] if read_z else None
        i += 1 if read_z else 0
        z_out = refs[i] if write_z else None
        i += 1 if write_z else 0
        (p_out, q_out, idx_s, idx_d, ra0, ra1, rb0, rb1, zb0, zb1,
         p_sh, q_sh, gi0, gi1, gz0, gz1, si0, si1, sz0, sz1) = refs[i:]
        RA = (ra0, ra1)
        RB = (rb0, rb1)
        ZB = (zb0, zb1)
        GI = (gi0, gi1)
        GZ = (gz0, gz1)
        SI = (si0, si1)
        SZ = (sz0, sz1)
        c, s, wid = _tile_ids()
        pltpu.sync_copy(src_h.at[pl.ds(wid * IROWS, IROWS)], idx_s)
        pltpu.sync_copy(dst_h.at[pl.ds(wid * IROWS, IROWS)], idx_d)
        _fill(zb0, CH, D, 0.0)
        _zero_acc(zb0, p_sh, s, CH)
        _zero_acc(zb0, q_sh, s, CH)
        plsc.subcore_barrier()

        iota = lax.iota(I32, 16)
        ebase = wid * EPT

        def gathers(t, p, fire):
            cps = []
            for u in range(TPS):
                r = t * TPS + u
                cps.append(pltpu.make_async_copy(
                    xbs_h.at[idx_s.at[r]],
                    RA[p].at[pl.ds(u * SUB, SUB)], GI[p]))
                cps.append(pltpu.make_async_copy(
                    xbd_h.at[idx_d.at[r]],
                    RB[p].at[pl.ds(u * SUB, SUB)], GI[p]))
            if read_z:
                cps.append(pltpu.make_async_copy(
                    z_in.at[pl.ds(ebase + t * CH, CH)], ZB[p], GZ[p]))
            for cp in cps:
                cp.start() if fire else cp.wait()

        def scatters(t, p, fire):
            if write_z:
                cp = pltpu.make_async_copy(
                    ZB[p], z_out.at[pl.ds(ebase + t * CH, CH)], SZ[p])
                cp.start() if fire else cp.wait()
            for u in range(TPS):
                r = t * TPS + u
                cp = pltpu.make_async_copy(
                    ZB[p].at[pl.ds(u * SUB, SUB)], p_sh.at[idx_s.at[r]], SI[p])
                cp.start(add=True) if fire else cp.wait()
                cp = pltpu.make_async_copy(
                    ZB[p].at[pl.ds(u * SUB, SUB)], q_sh.at[idx_d.at[r]], SI[p])
                cp.start(add=True) if fire else cp.wait()

        def compute(p):
            UNROLL = 8

            def group(g, gcarry):
                base = g * UNROLL
                for rr in range(UNROLL):
                    row = base + rr
                    halves = []
                    ssum = None
                    for hh in range(2):
                        sl = pl.ds(hh * 16, 16)
                        a = RA[p][row, sl]
                        b = RB[p][row, sl]
                        zb = BETA * (a - b)
                        if read_z:
                            zb = zb + ZB[p][row, sl]
                        sq = zb * zb
                        ssum = sq if ssum is None else ssum + sq
                        halves.append(zb)
                    sv = lax.reduce_sum_p.bind(ssum, axes=(0,))
                    over = sv > LAM * LAM
                    r = _nrsqrt_scalar(jnp.maximum(sv, LAM * LAM))
                    scale = jnp.where(over, LAM * r, 1.0)
                    for hh in range(2):
                        ZB[p][row, pl.ds(hh * 16, 16)] = halves[hh] * scale
                return gcarry

            lax.fori_loop(0, CH // UNROLL, group, 0)

        gathers(0, 0, True)
        gathers(1, 1, True)

        def step(k, carry):
            t0 = 2 * k
            gathers(t0, 0, False)
            compute(0)
            scatters(t0, 0, True)
            gathers(t0 + 1, 1, False)
            compute(1)
            scatters(t0 + 1, 1, True)
            scatters(t0, 0, False)
            gathers(t0 + 2, 0, True)
            scatters(t0 + 1, 1, False)
            gathers(t0 + 3, 1, True)
            return carry

        lax.fori_loop(0, H // 2 - 1, step, 0)
        gathers(H - 2, 0, False)
        compute(0)
        scatters(H - 2, 0, True)
        gathers(H - 1, 1, False)
        compute(1)
        scatters(H - 1, 1, True)
        scatters(H - 2, 0, False)
        scatters(H - 1, 1, False)
        plsc.subcore_barrier()
        rb = s * RPT
        pltpu.sync_copy(p_sh.at[pl.ds(rb, RPT)], p_out.at[c, pl.ds(rb, RPT)])
        pltpu.sync_copy(q_sh.at[pl.ds(rb, RPT)], q_out.at[c, pl.ds(rb, RPT)])

    return pl.kernel(body, out_type=out_type, mesh=_mesh(),
                     scratch_types=scratch, name="sc_dual",
                     compiler_params=_SC_PARAMS)


def _mlp(feat, W1, b1, W2, b2):
    def body(f_ref, w1_ref, b1_ref, w2_ref, b2_ref, o_ref):
        h1 = jnp.dot(f_ref[...], w1_ref[...], preferred_element_type=F32)
        h1 = jnp.maximum(h1 + b1_ref[...], 0.0)
        o_ref[...] = jnp.dot(h1, w2_ref[...],
                             preferred_element_type=F32) + b2_ref[...]

    return pl.pallas_call(
        body,
        grid=(10,),
        in_specs=[
            pl.BlockSpec((1000, 128), lambda i: (i, 0)),
            pl.BlockSpec((128, 64), lambda i: (0, 0)),
            pl.BlockSpec((1, 64), lambda i: (0, 0)),
            pl.BlockSpec((64, 32), lambda i: (0, 0)),
            pl.BlockSpec((1, 32), lambda i: (0, 0)),
        ],
        out_specs=pl.BlockSpec((1000, 32), lambda i: (i, 0)),
        out_shape=jax.ShapeDtypeStruct((N, D), F32),
    )(feat, W1, b1.reshape(1, 64), W2, b2.reshape(1, 32))


_NB = 2528


def _nspec(shape3=False, width=D):
    if shape3:
        return pl.BlockSpec((NC, _NB, width), lambda i: (0, i, 0))
    return pl.BlockSpec((_NB, width), lambda i: (i, 0))


def _nshape():
    return jax.ShapeDtypeStruct((NP, D), F32)


def _prep(dego, degi, h_pad):
    def body(do_ref, di_ref, h_ref, dob_ref, dib_ref, xs_ref):
        dso = do_ref[0, :, 0:1] + do_ref[1, :, 0:1]
        dsi = di_ref[0, :, 0:1] + di_ref[1, :, 0:1]
        dob = jnp.broadcast_to(lax.rsqrt(jnp.maximum(dso, 1.0)), (_NB, D))
        dib = jnp.broadcast_to(lax.rsqrt(jnp.maximum(dsi, 1.0)), (_NB, D))
        dob_ref[...] = dob
        dib_ref[...] = dib
        xs_ref[...] = dob * h_ref[...]

    return pl.pallas_call(
        body,
        grid=(NP // _NB,),
        in_specs=[_nspec(True, 16), _nspec(True, 16), _nspec()],
        out_specs=[_nspec(), _nspec(), _nspec()],
        out_shape=[_nshape(), _nshape(), _nshape()],
    )(dego, degi, h_pad)


def _node1(A, h_pad, dob, dib, P=None, Q=None):
    have_pq = P is not None

    def body(*refs):
        a_ref, h_ref, dob_ref, dib_ref = refs[:4]
        i = 4
        if have_pq:
            p_ref, q_ref = refs[i], refs[i + 1]
            i += 2
        y_ref, xbs_ref, xbd_ref = refs[i:]
        dob_v, dib_v = dob_ref[...], dib_ref[...]
        y = GAMMA * h_ref[...] + (1.0 - GAMMA) * dib_v * (
            a_ref[0] + a_ref[1])
        xbar = y
        if have_pq:
            u = dob_v * (p_ref[0] + p_ref[1]) - dib_v * (q_ref[0] + q_ref[1])
            xbar = y - GAMMA * u
        y_ref[...] = y
        xbs_ref[...] = dob_v * xbar
        xbd_ref[...] = dib_v * xbar

    in_specs = [_nspec(True), _nspec(), _nspec(), _nspec()]
    args = [A, h_pad, dob, dib]
    if have_pq:
        in_specs += [_nspec(True), _nspec(True)]
        args += [P, Q]
    return pl.pallas_call(
        body,
        grid=(NP // _NB,),
        in_specs=in_specs,
        out_specs=[_nspec(), _nspec(), _nspec()],
        out_shape=[_nshape(), _nshape(), _nshape()],
    )(*args)


def _node2(y, P, Q, dob, dib, want_xs):
    def body(*refs):
        y_ref, p_ref, q_ref, dob_ref, dib_ref = refs[:5]
        outs = refs[5:]
        dob_v, dib_v = dob_ref[...], dib_ref[...]
        u = dob_v * (p_ref[0] + p_ref[1]) - dib_v * (q_ref[0] + q_ref[1])
        x = y_ref[...] - GAMMA * u
        outs[0][...] = x
        if want_xs:
            outs[1][...] = dob_v * x

    n_out = 2 if want_xs else 1
    return pl.pallas_call(
        body,
        grid=(NP // _NB,),
        in_specs=[_nspec(), _nspec(True), _nspec(True), _nspec(), _nspec()],
        out_specs=[_nspec()] * n_out,
        out_shape=[_nshape()] * n_out,
    )(y, P, Q, dob, dib)


@jax.jit
def kernel(feat, edge_index, W1, b1, W2, b2):
    src = edge_index[0]
    dst = edge_index[1]
    pad = jnp.full((EP - E,), N, I32)
    src_p = jnp.concatenate([src, pad]).reshape(EP // SUB, SUB)
    dst_p = jnp.concatenate([dst, pad]).reshape(EP // SUB, SUB)

    h = _mlp(feat, W1, b1, W2, b2)
    h_pad = jnp.pad(h, ((0, NP - N), (0, 0)))

    dego, degi = _deg_kernel()(src_p, dst_p)
    dob, dib, xs = _prep(dego, degi, h_pad)

    pass_a = _pass_a_kernel()
    pass_b_first = _pass_b_kernel(read_z=False, write_z=True)
    pass_b_mid = _pass_b_kernel(read_z=True, write_z=True)
    pass_b_last = _pass_b_kernel(read_z=True, write_z=False)

    A = pass_a(xs, src_p, dst_p)
    y, xbs, xbd = _node1(A, h_pad, dob, dib)
    z, P, Q = pass_b_first(xbs, xbd, src_p, dst_p)
    x, xs = _node2(y, P, Q, dob, dib, want_xs=True)

    A = pass_a(xs, src_p, dst_p)
    y, xbs, xbd = _node1(A, h_pad, dob, dib, P, Q)
    z, P, Q = pass_b_mid(xbs, xbd, src_p, dst_p, z)
    x, xs = _node2(y, P, Q, dob, dib, want_xs=True)

    A = pass_a(xs, src_p, dst_p)
    y, xbs, xbd = _node1(A, h_pad, dob, dib, P, Q)
    P, Q = pass_b_last(xbs, xbd, src_p, dst_p, z)
    (x,) = _node2(y, P, Q, dob, dib, want_xs=False)

    return x[:N]


# speedup vs baseline: 16.1670x; 1.1865x over previous
import jax
import jax.numpy as jnp
from jax import lax
from jax.experimental import pallas as pl
from jax.experimental.pallas import tpu as pltpu
from jax.experimental.pallas import tpu_sc as plsc

N = 10000
NP = 10112
E = 320000
D = 32
NC = 2
NS = 16
NW = NC * NS
SUB = 128
EPT = 10240
EP = EPT * NW
IROWS = EPT // SUB
RPT = NP // NS
GAMMA = 0.25
BETA = 2.0
LAM = 3.0

F32 = jnp.float32
I32 = jnp.int32

_SC_PARAMS = pltpu.CompilerParams(use_tc_tiling_on_sc=False,
                                  needs_layout_passes=False)


def _mesh():
    return plsc.VectorSubcoreMesh(core_axis_name="c", subcore_axis_name="s",
                                  num_cores=NC, num_subcores=NS)


def _fill(ref, rows, width, value):
    v = jnp.full((16,), value, F32)

    def body(i, carry):
        for k in range(width // 16):
            ref[i, pl.ds(k * 16, 16)] = v
        return carry

    lax.fori_loop(0, rows, body, 0)


def _zero_acc(zsrc, acc_sh, s, ch):
    base = s * RPT
    off = 0
    while off < RPT:
        ln = min(ch, RPT - off)
        pltpu.sync_copy(zsrc.at[pl.ds(0, ln)], acc_sh.at[pl.ds(base + off, ln)])
        off += ln


def _nrsqrt(s):
    i = plsc.bitcast(s, I32)
    i = jnp.int32(0x5F3759DF) - lax.shift_right_logical(i, 1)
    y = plsc.bitcast(i, F32)
    for _ in range(3):
        y = y * (1.5 - 0.5 * s * y * y)
    return y


def _nrsqrt_scalar(s):
    i = lax.bitcast_convert_type(s, I32)
    i = jnp.int32(0x5F3759DF) - lax.shift_right_logical(i, 1)
    y = lax.bitcast_convert_type(i, F32)
    for _ in range(3):
        y = y * (1.5 - 0.5 * s * y * y)
    return y


def _tile_ids():
    c = lax.axis_index("c")
    s = lax.axis_index("s")
    return c, s, c * NS + s


def _deg_kernel():
    NSUB = 8
    scratch = [
        pltpu.VMEM((NSUB, SUB), I32),
        pltpu.VMEM((NSUB, SUB), I32),
        pltpu.VMEM((SUB, 16), F32),
        pltpu.VMEM((RPT, 16), F32),
        pltpu.VMEM_SHARED((NP, 16), F32),
        pltpu.VMEM_SHARED((NP, 16), F32),
    ]
    out_type = (jax.ShapeDtypeStruct((NC, NP, 16), F32),
                jax.ShapeDtypeStruct((NC, NP, 16), F32))

    def body(src_h, dst_h, do_out, di_out, idx_s, idx_d, ones_b, zstage,
             do_sh, di_sh):
        c, s, wid = _tile_ids()
        _fill(ones_b, SUB, 16, 1.0)
        _fill(zstage, RPT, 16, 0.0)
        pltpu.sync_copy(zstage, do_sh.at[pl.ds(s * RPT, RPT)])
        pltpu.sync_copy(zstage, di_sh.at[pl.ds(s * RPT, RPT)])
        plsc.subcore_barrier()

        def chunk(ch, carry):
            rb = wid * IROWS + ch * NSUB
            pltpu.sync_copy(src_h.at[pl.ds(rb, NSUB)], idx_s)
            pltpu.sync_copy(dst_h.at[pl.ds(rb, NSUB)], idx_d)
            for j in range(NSUB):
                pltpu.sync_copy(ones_b, do_sh.at[idx_s.at[j]], add=True)
                pltpu.sync_copy(ones_b, di_sh.at[idx_d.at[j]], add=True)
            return carry

        lax.fori_loop(0, IROWS // NSUB, chunk, 0)
        plsc.subcore_barrier()
        rb = s * RPT
        pltpu.sync_copy(do_sh.at[pl.ds(rb, RPT)], do_out.at[c, pl.ds(rb, RPT)])
        pltpu.sync_copy(di_sh.at[pl.ds(rb, RPT)], di_out.at[c, pl.ds(rb, RPT)])

    return pl.kernel(body, out_type=out_type, mesh=_mesh(),
                     scratch_types=scratch, name="sc_degrees",
                     compiler_params=_SC_PARAMS)


def _pass_a_kernel():
    CH = 512
    TPS = CH // SUB
    H = EPT // CH
    scratch = [
        pltpu.VMEM((IROWS, SUB), I32),
        pltpu.VMEM((IROWS, SUB), I32),
        pltpu.VMEM((CH, D), F32),
        pltpu.VMEM((CH, D), F32),
        pltpu.VMEM_SHARED((NP, D), F32),
        pltpu.VMEM_SHARED((NP, D), F32),
        pltpu.SemaphoreType.DMA,
        pltpu.SemaphoreType.DMA,
        pltpu.SemaphoreType.DMA,
        pltpu.SemaphoreType.DMA,
    ]
    out_type = jax.ShapeDtypeStruct((NC, NP, D), F32)

    def body(xs_h, src_h, dst_h, a_out, idx_s, idx_d, r0, r1, a_sh, xs_sh,
             g0, g1, s0, s1):
        ROWS = (r0, r1)
        GS = (g0, g1)
        SS = (s0, s1)
        c, s, wid = _tile_ids()
        pltpu.sync_copy(src_h.at[pl.ds(wid * IROWS, IROWS)], idx_s)
        pltpu.sync_copy(dst_h.at[pl.ds(wid * IROWS, IROWS)], idx_d)
        _fill(r0, CH, D, 0.0)
        _zero_acc(r0, a_sh, s, CH)
        rb0 = s * RPT
        pltpu.sync_copy(xs_h.at[pl.ds(rb0, RPT)], xs_sh.at[pl.ds(rb0, RPT)])
        plsc.subcore_barrier()

        def gathers(t, p, fire):
            for u in range(TPS):
                cp = pltpu.make_async_copy(
                    xs_sh.at[idx_s.at[t * TPS + u]],
                    ROWS[p].at[pl.ds(u * SUB, SUB)], GS[p])
                cp.start() if fire else cp.wait()

        def scatters(t, p, fire):
            for u in range(TPS):
                cp = pltpu.make_async_copy(
                    ROWS[p].at[pl.ds(u * SUB, SUB)],
                    a_sh.at[idx_d.at[t * TPS + u]], SS[p])
                cp.start(add=True) if fire else cp.wait()

        gathers(0, 0, True)
        gathers(1, 1, True)

        def step(k, carry):
            t0 = 2 * k
            gathers(t0, 0, False)
            scatters(t0, 0, True)
            gathers(t0 + 1, 1, False)
            scatters(t0 + 1, 1, True)
            scatters(t0, 0, False)
            gathers(t0 + 2, 0, True)
            scatters(t0 + 1, 1, False)
            gathers(t0 + 3, 1, True)
            return carry

        lax.fori_loop(0, H // 2 - 1, step, 0)
        gathers(H - 2, 0, False)
        scatters(H - 2, 0, True)
        gathers(H - 1, 1, False)
        scatters(H - 1, 1, True)
        scatters(H - 2, 0, False)
        scatters(H - 1, 1, False)
        plsc.subcore_barrier()
        rb = s * RPT
        pltpu.sync_copy(a_sh.at[pl.ds(rb, RPT)], a_out.at[c, pl.ds(rb, RPT)])

    return pl.kernel(body, out_type=out_type, mesh=_mesh(),
                     scratch_types=scratch, name="sc_adj",
                     compiler_params=_SC_PARAMS)


def _pass_b_kernel(read_z, write_z):
    CH = 256
    TPS = CH // SUB
    H = EPT // CH
    scratch = [
        pltpu.VMEM((IROWS, SUB), I32),
        pltpu.VMEM((IROWS, SUB), I32),
        pltpu.VMEM((CH, D), F32),
        pltpu.VMEM((CH, D), F32),
        pltpu.VMEM((CH, D), F32),
        pltpu.VMEM((CH, D), F32),
        pltpu.VMEM((CH, D), F32),
        pltpu.VMEM((CH, D), F32),
        pltpu.VMEM_SHARED((NP, D), F32),
        pltpu.VMEM_SHARED((NP, D), F32),
        pltpu.SemaphoreType.DMA,
        pltpu.SemaphoreType.DMA,
        pltpu.SemaphoreType.DMA,
        pltpu.SemaphoreType.DMA,
        pltpu.SemaphoreType.DMA,
        pltpu.SemaphoreType.DMA,
        pltpu.SemaphoreType.DMA,
        pltpu.SemaphoreType.DMA,
    ]
    out_type = ([jax.ShapeDtypeStruct((EP, D), F32)] if write_z else []) + [
        jax.ShapeDtypeStruct((NC, NP, D), F32),
        jax.ShapeDtypeStruct((NC, NP, D), F32)]
    out_type = tuple(out_type)

    def body(*refs):
        xbs_h, xbd_h, src_h, dst_h = refs[:4]
        i = 4
        z_in = refs[i] if read_z else None
        i += 1 if read_z else 0
        z_out = refs[i] if write_z else None
        i += 1 if write_z else 0
        (p_out, q_out, idx_s, idx_d, ra0, ra1, rb0, rb1, zb0, zb1,
         p_sh, q_sh, gi0, gi1, gz0, gz1, si0, si1, sz0, sz1) = refs[i:]
        RA = (ra0, ra1)
        RB = (rb0, rb1)
        ZB = (zb0, zb1)
        GI = (gi0, gi1)
        GZ = (gz0, gz1)
        SI = (si0, si1)
        SZ = (sz0, sz1)
        c, s, wid = _tile_ids()
        pltpu.sync_copy(src_h.at[pl.ds(wid * IROWS, IROWS)], idx_s)
        pltpu.sync_copy(dst_h.at[pl.ds(wid * IROWS, IROWS)], idx_d)
        _fill(zb0, CH, D, 0.0)
        _zero_acc(zb0, p_sh, s, CH)
        _zero_acc(zb0, q_sh, s, CH)
        plsc.subcore_barrier()

        iota = lax.iota(I32, 16)
        ebase = wid * EPT

        def gathers(t, p, fire):
            cps = []
            for u in range(TPS):
                r = t * TPS + u
                cps.append(pltpu.make_async_copy(
                    xbs_h.at[idx_s.at[r]],
                    RA[p].at[pl.ds(u * SUB, SUB)], GI[p]))
                cps.append(pltpu.make_async_copy(
                    xbd_h.at[idx_d.at[r]],
                    RB[p].at[pl.ds(u * SUB, SUB)], GI[p]))
            if read_z:
                cps.append(pltpu.make_async_copy(
                    z_in.at[pl.ds(ebase + t * CH, CH)], ZB[p], GZ[p]))
            for cp in cps:
                cp.start() if fire else cp.wait()

        def scatters(t, p, fire):
            if write_z:
                cp = pltpu.make_async_copy(
                    ZB[p], z_out.at[pl.ds(ebase + t * CH, CH)], SZ[p])
                cp.start() if fire else cp.wait()
            for u in range(TPS):
                r = t * TPS + u
                cp = pltpu.make_async_copy(
                    ZB[p].at[pl.ds(u * SUB, SUB)], p_sh.at[idx_s.at[r]], SI[p])
                cp.start(add=True) if fire else cp.wait()
                cp = pltpu.make_async_copy(
                    ZB[p].at[pl.ds(u * SUB, SUB)], q_sh.at[idx_d.at[r]], SI[p])
                cp.start(add=True) if fire else cp.wait()

        def compute(p):
            UNROLL = 8

            def group(g, gcarry):
                base = g * UNROLL
                for rr in range(UNROLL):
                    row = base + rr
                    halves = []
                    ssum = None
                    for hh in range(2):
                        sl = pl.ds(hh * 16, 16)
                        a = RA[p][row, sl]
                        b = RB[p][row, sl]
                        zb = BETA * (a - b)
                        if read_z:
                            zb = zb + ZB[p][row, sl]
                        sq = zb * zb
                        ssum = sq if ssum is None else ssum + sq
                        halves.append(zb)
                    sv = lax.reduce_sum_p.bind(ssum, axes=(0,))
                    over = sv > LAM * LAM
                    r = _nrsqrt_scalar(jnp.maximum(sv, LAM * LAM))
                    scale = jnp.where(over, LAM * r, 1.0)
                    for hh in range(2):
                        ZB[p][row, pl.ds(hh * 16, 16)] = halves[hh] * scale
                return gcarry

            lax.fori_loop(0, CH // UNROLL, group, 0)

        gathers(0, 0, True)
        gathers(1, 1, True)

        def step(k, carry):
            t0 = 2 * k
            gathers(t0, 0, False)
            compute(0)
            scatters(t0, 0, True)
            gathers(t0 + 1, 1, False)
            compute(1)
            scatters(t0 + 1, 1, True)
            scatters(t0, 0, False)
            gathers(t0 + 2, 0, True)
            scatters(t0 + 1, 1, False)
            gathers(t0 + 3, 1, True)
            return carry

        lax.fori_loop(0, H // 2 - 1, step, 0)
        gathers(H - 2, 0, False)
        compute(0)
        scatters(H - 2, 0, True)
        gathers(H - 1, 1, False)
        compute(1)
        scatters(H - 1, 1, True)
        scatters(H - 2, 0, False)
        scatters(H - 1, 1, False)
        plsc.subcore_barrier()
        rb = s * RPT
        pltpu.sync_copy(p_sh.at[pl.ds(rb, RPT)], p_out.at[c, pl.ds(rb, RPT)])
        pltpu.sync_copy(q_sh.at[pl.ds(rb, RPT)], q_out.at[c, pl.ds(rb, RPT)])

    return pl.kernel(body, out_type=out_type, mesh=_mesh(),
                     scratch_types=scratch, name="sc_dual",
                     compiler_params=_SC_PARAMS)


def _mlp(feat, W1, b1, W2, b2):
    def body(f_ref, w1_ref, b1_ref, w2_ref, b2_ref, o_ref):
        h1 = jnp.dot(f_ref[...], w1_ref[...], preferred_element_type=F32)
        h1 = jnp.maximum(h1 + b1_ref[...], 0.0)
        o_ref[...] = jnp.dot(h1, w2_ref[...],
                             preferred_element_type=F32) + b2_ref[...]

    return pl.pallas_call(
        body,
        grid=(10,),
        in_specs=[
            pl.BlockSpec((1000, 128), lambda i: (i, 0)),
            pl.BlockSpec((128, 64), lambda i: (0, 0)),
            pl.BlockSpec((1, 64), lambda i: (0, 0)),
            pl.BlockSpec((64, 32), lambda i: (0, 0)),
            pl.BlockSpec((1, 32), lambda i: (0, 0)),
        ],
        out_specs=pl.BlockSpec((1000, 32), lambda i: (i, 0)),
        out_shape=jax.ShapeDtypeStruct((N, D), F32),
    )(feat, W1, b1.reshape(1, 64), W2, b2.reshape(1, 32))


_NB = 2528


def _nspec(shape3=False, width=D):
    if shape3:
        return pl.BlockSpec((NC, _NB, width), lambda i: (0, i, 0))
    return pl.BlockSpec((_NB, width), lambda i: (i, 0))


def _nshape():
    return jax.ShapeDtypeStruct((NP, D), F32)


def _prep(dego, degi, h_pad):
    def body(do_ref, di_ref, h_ref, dob_ref, dib_ref, xs_ref):
        dso = do_ref[0, :, 0:1] + do_ref[1, :, 0:1]
        dsi = di_ref[0, :, 0:1] + di_ref[1, :, 0:1]
        dob = jnp.broadcast_to(lax.rsqrt(jnp.maximum(dso, 1.0)), (_NB, D))
        dib = jnp.broadcast_to(lax.rsqrt(jnp.maximum(dsi, 1.0)), (_NB, D))
        dob_ref[...] = dob
        dib_ref[...] = dib
        xs_ref[...] = dob * h_ref[...]

    return pl.pallas_call(
        body,
        grid=(NP // _NB,),
        in_specs=[_nspec(True, 16), _nspec(True, 16), _nspec()],
        out_specs=[_nspec(), _nspec(), _nspec()],
        out_shape=[_nshape(), _nshape(), _nshape()],
    )(dego, degi, h_pad)


def _node1(A, h_pad, dob, dib, P=None, Q=None):
    have_pq = P is not None

    def body(*refs):
        a_ref, h_ref, dob_ref, dib_ref = refs[:4]
        i = 4
        if have_pq:
            p_ref, q_ref = refs[i], refs[i + 1]
            i += 2
        y_ref, xbs_ref, xbd_ref = refs[i:]
        dob_v, dib_v = dob_ref[...], dib_ref[...]
        y = GAMMA * h_ref[...] + (1.0 - GAMMA) * dib_v * (
            a_ref[0] + a_ref[1])
        xbar = y
        if have_pq:
            u = dob_v * (p_ref[0] + p_ref[1]) - dib_v * (q_ref[0] + q_ref[1])
            xbar = y - GAMMA * u
        y_ref[...] = y
        xbs_ref[...] = dob_v * xbar
        xbd_ref[...] = dib_v * xbar

    in_specs = [_nspec(True), _nspec(), _nspec(), _nspec()]
    args = [A, h_pad, dob, dib]
    if have_pq:
        in_specs += [_nspec(True), _nspec(True)]
        args += [P, Q]
    return pl.pallas_call(
        body,
        grid=(NP // _NB,),
        in_specs=in_specs,
        out_specs=[_nspec(), _nspec(), _nspec()],
        out_shape=[_nshape(), _nshape(), _nshape()],
    )(*args)


def _node2(y, P, Q, dob, dib, want_xs):
    def body(*refs):
        y_ref, p_ref, q_ref, dob_ref, dib_ref = refs[:5]
        outs = refs[5:]
        dob_v, dib_v = dob_ref[...], dib_ref[...]
        u = dob_v * (p_ref[0] + p_ref[1]) - dib_v * (q_ref[0] + q_ref[1])
        x = y_ref[...] - GAMMA * u
        outs[0][...] = x
        if want_xs:
            outs[1][...] = dob_v * x

    n_out = 2 if want_xs else 1
    return pl.pallas_call(
        body,
        grid=(NP // _NB,),
        in_specs=[_nspec(), _nspec(True), _nspec(True), _nspec(), _nspec()],
        out_specs=[_nspec()] * n_out,
        out_shape=[_nshape()] * n_out,
    )(y, P, Q, dob, dib)


@jax.jit
def kernel(feat, edge_index, W1, b1, W2, b2):
    src = edge_index[0]
    dst = edge_index[1]
    pad = jnp.full((EP - E,), N, I32)
    src_p = jnp.concatenate([src, pad]).reshape(EP // SUB, SUB)
    dst_p = jnp.concatenate([dst, pad]).reshape(EP // SUB, SUB)

    h = _mlp(feat, W1, b1, W2, b2)
    h_pad = jnp.pad(h, ((0, NP - N), (0, 0)))

    dego, degi = _deg_kernel()(src_p, dst_p)
    dob, dib, xs = _prep(dego, degi, h_pad)

    pass_a = _pass_a_kernel()
    pass_b_first = _pass_b_kernel(read_z=False, write_z=True)
    pass_b_mid = _pass_b_kernel(read_z=True, write_z=True)
    pass_b_last = _pass_b_kernel(read_z=True, write_z=False)

    A = pass_a(xs, src_p, dst_p)
    y, xbs, xbd = _node1(A, h_pad, dob, dib)
    z, P, Q = pass_b_first(xbs, xbd, src_p, dst_p)
    x, xs = _node2(y, P, Q, dob, dib, want_xs=True)

    A = pass_a(xs, src_p, dst_p)
    y, xbs, xbd = _node1(A, h_pad, dob, dib, P, Q)
    z, P, Q = pass_b_mid(xbs, xbd, src_p, dst_p, z)
    x, xs = _node2(y, P, Q, dob, dib, want_xs=True)

    A = pass_a(xs, src_p, dst_p)
    y, xbs, xbd = _node1(A, h_pad, dob, dib, P, Q)
    P, Q = pass_b_last(xbs, xbd, src_p, dst_p, z)
    (x,) = _node2(y, P, Q, dob, dib, want_xs=False)

    return x[:N]


# speedup vs baseline: 20.8921x; 1.2923x over previous
import jax
import jax.numpy as jnp
from jax import lax
from jax.experimental import pallas as pl
from jax.experimental.pallas import tpu as pltpu
from jax.experimental.pallas import tpu_sc as plsc

N = 10000
NP = 10112
E = 320000
D = 32
NC = 2
NS = 16
NW = NC * NS
SUB = 128
EPT = 10240
EP = EPT * NW
IROWS = EPT // SUB
RPT = NP // NS
GAMMA = 0.25
BETA = 2.0
LAM = 3.0

F32 = jnp.float32
I32 = jnp.int32

_SC_PARAMS = pltpu.CompilerParams(use_tc_tiling_on_sc=False,
                                  needs_layout_passes=False)


def _mesh():
    return plsc.VectorSubcoreMesh(core_axis_name="c", subcore_axis_name="s",
                                  num_cores=NC, num_subcores=NS)


def _fill(ref, rows, width, value):
    v = jnp.full((16,), value, F32)

    def body(i, carry):
        for k in range(width // 16):
            ref[i, pl.ds(k * 16, 16)] = v
        return carry

    lax.fori_loop(0, rows, body, 0)


def _zero_acc(zsrc, acc_sh, s, ch):
    base = s * RPT
    off = 0
    while off < RPT:
        ln = min(ch, RPT - off)
        pltpu.sync_copy(zsrc.at[pl.ds(0, ln)], acc_sh.at[pl.ds(base + off, ln)])
        off += ln


def _nrsqrt(s):
    i = plsc.bitcast(s, I32)
    i = jnp.int32(0x5F3759DF) - lax.shift_right_logical(i, 1)
    y = plsc.bitcast(i, F32)
    for _ in range(3):
        y = y * (1.5 - 0.5 * s * y * y)
    return y


def _nrsqrt_scalar(s):
    i = lax.bitcast_convert_type(s, I32)
    i = jnp.int32(0x5F3759DF) - lax.shift_right_logical(i, 1)
    y = lax.bitcast_convert_type(i, F32)
    for _ in range(3):
        y = y * (1.5 - 0.5 * s * y * y)
    return y


def _tile_ids():
    c = lax.axis_index("c")
    s = lax.axis_index("s")
    return c, s, c * NS + s


def _deg_kernel():
    NSUB = 8
    scratch = [
        pltpu.VMEM((NSUB, SUB), I32),
        pltpu.VMEM((NSUB, SUB), I32),
        pltpu.VMEM((SUB, 16), F32),
        pltpu.VMEM((RPT, 16), F32),
        pltpu.VMEM_SHARED((NP, 16), F32),
        pltpu.VMEM_SHARED((NP, 16), F32),
    ]
    out_type = (jax.ShapeDtypeStruct((NC, NP, 16), F32),
                jax.ShapeDtypeStruct((NC, NP, 16), F32))

    def body(src_h, dst_h, do_out, di_out, idx_s, idx_d, ones_b, zstage,
             do_sh, di_sh):
        c, s, wid = _tile_ids()
        _fill(ones_b, SUB, 16, 1.0)
        _fill(zstage, RPT, 16, 0.0)
        pltpu.sync_copy(zstage, do_sh.at[pl.ds(s * RPT, RPT)])
        pltpu.sync_copy(zstage, di_sh.at[pl.ds(s * RPT, RPT)])
        plsc.subcore_barrier()

        def chunk(ch, carry):
            rb = wid * IROWS + ch * NSUB
            pltpu.sync_copy(src_h.at[pl.ds(rb, NSUB)], idx_s)
            pltpu.sync_copy(dst_h.at[pl.ds(rb, NSUB)], idx_d)
            for j in range(NSUB):
                pltpu.sync_copy(ones_b, do_sh.at[idx_s.at[j]], add=True)
                pltpu.sync_copy(ones_b, di_sh.at[idx_d.at[j]], add=True)
            return carry

        lax.fori_loop(0, IROWS // NSUB, chunk, 0)
        plsc.subcore_barrier()
        rb = s * RPT
        pltpu.sync_copy(do_sh.at[pl.ds(rb, RPT)], do_out.at[c, pl.ds(rb, RPT)])
        pltpu.sync_copy(di_sh.at[pl.ds(rb, RPT)], di_out.at[c, pl.ds(rb, RPT)])

    return pl.kernel(body, out_type=out_type, mesh=_mesh(),
                     scratch_types=scratch, name="sc_degrees",
                     compiler_params=_SC_PARAMS)


def _pass_a_kernel():
    CH = 512
    TPS = CH // SUB
    H = EPT // CH
    scratch = [
        pltpu.VMEM((IROWS, SUB), I32),
        pltpu.VMEM((IROWS, SUB), I32),
        pltpu.VMEM((CH, D), F32),
        pltpu.VMEM((CH, D), F32),
        pltpu.VMEM_SHARED((NP, D), F32),
        pltpu.VMEM_SHARED((NP, D), F32),
        pltpu.SemaphoreType.DMA,
        pltpu.SemaphoreType.DMA,
        pltpu.SemaphoreType.DMA,
        pltpu.SemaphoreType.DMA,
    ]
    out_type = jax.ShapeDtypeStruct((NC, NP, D), F32)

    def body(xs_h, src_h, dst_h, a_out, idx_s, idx_d, r0, r1, a_sh, xs_sh,
             g0, g1, s0, s1):
        ROWS = (r0, r1)
        GS = (g0, g1)
        SS = (s0, s1)
        c, s, wid = _tile_ids()
        pltpu.sync_copy(src_h.at[pl.ds(wid * IROWS, IROWS)], idx_s)
        pltpu.sync_copy(dst_h.at[pl.ds(wid * IROWS, IROWS)], idx_d)
        _fill(r0, CH, D, 0.0)
        _zero_acc(r0, a_sh, s, CH)
        rb0 = s * RPT
        pltpu.sync_copy(xs_h.at[pl.ds(rb0, RPT)], xs_sh.at[pl.ds(rb0, RPT)])
        plsc.subcore_barrier()

        def gathers(t, p, fire):
            for u in range(TPS):
                cp = pltpu.make_async_copy(
                    xs_sh.at[idx_s.at[t * TPS + u]],
                    ROWS[p].at[pl.ds(u * SUB, SUB)], GS[p])
                cp.start() if fire else cp.wait()

        def scatters(t, p, fire):
            for u in range(TPS):
                cp = pltpu.make_async_copy(
                    ROWS[p].at[pl.ds(u * SUB, SUB)],
                    a_sh.at[idx_d.at[t * TPS + u]], SS[p])
                cp.start(add=True) if fire else cp.wait()

        gathers(0, 0, True)
        gathers(1, 1, True)

        def step(k, carry):
            t0 = 2 * k
            gathers(t0, 0, False)
            scatters(t0, 0, True)
            gathers(t0 + 1, 1, False)
            scatters(t0 + 1, 1, True)
            scatters(t0, 0, False)
            gathers(t0 + 2, 0, True)
            scatters(t0 + 1, 1, False)
            gathers(t0 + 3, 1, True)
            return carry

        lax.fori_loop(0, H // 2 - 1, step, 0)
        gathers(H - 2, 0, False)
        scatters(H - 2, 0, True)
        gathers(H - 1, 1, False)
        scatters(H - 1, 1, True)
        scatters(H - 2, 0, False)
        scatters(H - 1, 1, False)
        plsc.subcore_barrier()
        rb = s * RPT
        pltpu.sync_copy(a_sh.at[pl.ds(rb, RPT)], a_out.at[c, pl.ds(rb, RPT)])

    return pl.kernel(body, out_type=out_type, mesh=_mesh(),
                     scratch_types=scratch, name="sc_adj",
                     compiler_params=_SC_PARAMS)


def _pass_b_kernel(read_z, write_z):
    CH = 128
    TPS = CH // SUB
    H = EPT // CH
    scratch = [
        pltpu.VMEM((IROWS, SUB), I32),
        pltpu.VMEM((IROWS, SUB), I32),
        pltpu.VMEM((CH, D), F32),
        pltpu.VMEM((CH, D), F32),
        pltpu.VMEM((CH, D), F32),
        pltpu.VMEM((CH, D), F32),
        pltpu.VMEM((CH, D), F32),
        pltpu.VMEM((CH, D), F32),
        pltpu.VMEM_SHARED((NP, D), F32),
        pltpu.VMEM_SHARED((NP, D), F32),
        pltpu.VMEM_SHARED((NP, D), F32),
        pltpu.VMEM_SHARED((NP, D), F32),
        pltpu.SemaphoreType.DMA,
        pltpu.SemaphoreType.DMA,
        pltpu.SemaphoreType.DMA,
        pltpu.SemaphoreType.DMA,
        pltpu.SemaphoreType.DMA,
        pltpu.SemaphoreType.DMA,
        pltpu.SemaphoreType.DMA,
        pltpu.SemaphoreType.DMA,
    ]
    out_type = ([jax.ShapeDtypeStruct((EP, D), F32)] if write_z else []) + [
        jax.ShapeDtypeStruct((NC, NP, D), F32),
        jax.ShapeDtypeStruct((NC, NP, D), F32)]
    out_type = tuple(out_type)

    def body(*refs):
        xbs_h, xbd_h, src_h, dst_h = refs[:4]
        i = 4
        z_in = refs[i] if read_z else None
        i += 1 if read_z else 0
        z_out = refs[i] if write_z else None
        i += 1 if write_z else 0
        (p_out, q_out, idx_s, idx_d, ra0, ra1, rb0, rb1, zb0, zb1,
         p_sh, q_sh, xbs_sh, xbd_sh,
         gi0, gi1, gz0, gz1, si0, si1, sz0, sz1) = refs[i:]
        RA = (ra0, ra1)
        RB = (rb0, rb1)
        ZB = (zb0, zb1)
        GI = (gi0, gi1)
        GZ = (gz0, gz1)
        SI = (si0, si1)
        SZ = (sz0, sz1)
        c, s, wid = _tile_ids()
        pltpu.sync_copy(src_h.at[pl.ds(wid * IROWS, IROWS)], idx_s)
        pltpu.sync_copy(dst_h.at[pl.ds(wid * IROWS, IROWS)], idx_d)
        _fill(zb0, CH, D, 0.0)
        _zero_acc(zb0, p_sh, s, CH)
        _zero_acc(zb0, q_sh, s, CH)
        rb0_ = s * RPT
        pltpu.sync_copy(xbs_h.at[pl.ds(rb0_, RPT)], xbs_sh.at[pl.ds(rb0_, RPT)])
        pltpu.sync_copy(xbd_h.at[pl.ds(rb0_, RPT)], xbd_sh.at[pl.ds(rb0_, RPT)])
        plsc.subcore_barrier()

        iota = lax.iota(I32, 16)
        ebase = wid * EPT

        def gathers(t, p, fire):
            cps = []
            for u in range(TPS):
                r = t * TPS + u
                cps.append(pltpu.make_async_copy(
                    xbs_sh.at[idx_s.at[r]],
                    RA[p].at[pl.ds(u * SUB, SUB)], GI[p]))
                cps.append(pltpu.make_async_copy(
                    xbd_sh.at[idx_d.at[r]],
                    RB[p].at[pl.ds(u * SUB, SUB)], GI[p]))
            if read_z:
                cps.append(pltpu.make_async_copy(
                    z_in.at[pl.ds(ebase + t * CH, CH)], ZB[p], GZ[p]))
            for cp in cps:
                cp.start() if fire else cp.wait()

        def scatters(t, p, fire):
            if write_z:
                cp = pltpu.make_async_copy(
                    ZB[p], z_out.at[pl.ds(ebase + t * CH, CH)], SZ[p])
                cp.start() if fire else cp.wait()
            for u in range(TPS):
                r = t * TPS + u
                cp = pltpu.make_async_copy(
                    ZB[p].at[pl.ds(u * SUB, SUB)], p_sh.at[idx_s.at[r]], SI[p])
                cp.start(add=True) if fire else cp.wait()
                cp = pltpu.make_async_copy(
                    ZB[p].at[pl.ds(u * SUB, SUB)], q_sh.at[idx_d.at[r]], SI[p])
                cp.start(add=True) if fire else cp.wait()

        def compute(p):
            UNROLL = 8

            def group(g, gcarry):
                base = g * UNROLL
                for rr in range(UNROLL):
                    row = base + rr
                    halves = []
                    ssum = None
                    for hh in range(2):
                        sl = pl.ds(hh * 16, 16)
                        a = RA[p][row, sl]
                        b = RB[p][row, sl]
                        zb = BETA * (a - b)
                        if read_z:
                            zb = zb + ZB[p][row, sl]
                        sq = zb * zb
                        ssum = sq if ssum is None else ssum + sq
                        halves.append(zb)
                    sv = lax.reduce_sum_p.bind(ssum, axes=(0,))
                    over = sv > LAM * LAM
                    r = _nrsqrt_scalar(jnp.maximum(sv, LAM * LAM))
                    scale = jnp.where(over, LAM * r, 1.0)
                    for hh in range(2):
                        ZB[p][row, pl.ds(hh * 16, 16)] = halves[hh] * scale
                return gcarry

            lax.fori_loop(0, CH // UNROLL, group, 0)

        gathers(0, 0, True)
        gathers(1, 1, True)

        def step(k, carry):
            t0 = 2 * k
            gathers(t0, 0, False)
            compute(0)
            scatters(t0, 0, True)
            gathers(t0 + 1, 1, False)
            compute(1)
            scatters(t0 + 1, 1, True)
            scatters(t0, 0, False)
            gathers(t0 + 2, 0, True)
            scatters(t0 + 1, 1, False)
            gathers(t0 + 3, 1, True)
            return carry

        lax.fori_loop(0, H // 2 - 1, step, 0)
        gathers(H - 2, 0, False)
        compute(0)
        scatters(H - 2, 0, True)
        gathers(H - 1, 1, False)
        compute(1)
        scatters(H - 1, 1, True)
        scatters(H - 2, 0, False)
        scatters(H - 1, 1, False)
        plsc.subcore_barrier()
        rb = s * RPT
        pltpu.sync_copy(p_sh.at[pl.ds(rb, RPT)], p_out.at[c, pl.ds(rb, RPT)])
        pltpu.sync_copy(q_sh.at[pl.ds(rb, RPT)], q_out.at[c, pl.ds(rb, RPT)])

    return pl.kernel(body, out_type=out_type, mesh=_mesh(),
                     scratch_types=scratch, name="sc_dual",
                     compiler_params=_SC_PARAMS)


def _mlp(feat, W1, b1, W2, b2):
    def body(f_ref, w1_ref, b1_ref, w2_ref, b2_ref, o_ref):
        h1 = jnp.dot(f_ref[...], w1_ref[...], preferred_element_type=F32)
        h1 = jnp.maximum(h1 + b1_ref[...], 0.0)
        o_ref[...] = jnp.dot(h1, w2_ref[...],
                             preferred_element_type=F32) + b2_ref[...]

    return pl.pallas_call(
        body,
        grid=(10,),
        in_specs=[
            pl.BlockSpec((1000, 128), lambda i: (i, 0)),
            pl.BlockSpec((128, 64), lambda i: (0, 0)),
            pl.BlockSpec((1, 64), lambda i: (0, 0)),
            pl.BlockSpec((64, 32), lambda i: (0, 0)),
            pl.BlockSpec((1, 32), lambda i: (0, 0)),
        ],
        out_specs=pl.BlockSpec((1000, 32), lambda i: (i, 0)),
        out_shape=jax.ShapeDtypeStruct((N, D), F32),
    )(feat, W1, b1.reshape(1, 64), W2, b2.reshape(1, 32))


_NB = 2528


def _nspec(shape3=False, width=D):
    if shape3:
        return pl.BlockSpec((NC, _NB, width), lambda i: (0, i, 0))
    return pl.BlockSpec((_NB, width), lambda i: (i, 0))


def _nshape():
    return jax.ShapeDtypeStruct((NP, D), F32)


def _prep(dego, degi, h_pad):
    def body(do_ref, di_ref, h_ref, dob_ref, dib_ref, xs_ref):
        dso = do_ref[0, :, 0:1] + do_ref[1, :, 0:1]
        dsi = di_ref[0, :, 0:1] + di_ref[1, :, 0:1]
        dob = jnp.broadcast_to(lax.rsqrt(jnp.maximum(dso, 1.0)), (_NB, D))
        dib = jnp.broadcast_to(lax.rsqrt(jnp.maximum(dsi, 1.0)), (_NB, D))
        dob_ref[...] = dob
        dib_ref[...] = dib
        xs_ref[...] = dob * h_ref[...]

    return pl.pallas_call(
        body,
        grid=(NP // _NB,),
        in_specs=[_nspec(True, 16), _nspec(True, 16), _nspec()],
        out_specs=[_nspec(), _nspec(), _nspec()],
        out_shape=[_nshape(), _nshape(), _nshape()],
    )(dego, degi, h_pad)


def _node1(A, h_pad, dob, dib, P=None, Q=None):
    have_pq = P is not None

    def body(*refs):
        a_ref, h_ref, dob_ref, dib_ref = refs[:4]
        i = 4
        if have_pq:
            p_ref, q_ref = refs[i], refs[i + 1]
            i += 2
        y_ref, xbs_ref, xbd_ref = refs[i:]
        dob_v, dib_v = dob_ref[...], dib_ref[...]
        y = GAMMA * h_ref[...] + (1.0 - GAMMA) * dib_v * (
            a_ref[0] + a_ref[1])
        xbar = y
        if have_pq:
            u = dob_v * (p_ref[0] + p_ref[1]) - dib_v * (q_ref[0] + q_ref[1])
            xbar = y - GAMMA * u
        y_ref[...] = y
        xbs_ref[...] = dob_v * xbar
        xbd_ref[...] = dib_v * xbar

    in_specs = [_nspec(True), _nspec(), _nspec(), _nspec()]
    args = [A, h_pad, dob, dib]
    if have_pq:
        in_specs += [_nspec(True), _nspec(True)]
        args += [P, Q]
    return pl.pallas_call(
        body,
        grid=(NP // _NB,),
        in_specs=in_specs,
        out_specs=[_nspec(), _nspec(), _nspec()],
        out_shape=[_nshape(), _nshape(), _nshape()],
    )(*args)


def _node2(y, P, Q, dob, dib, want_xs):
    def body(*refs):
        y_ref, p_ref, q_ref, dob_ref, dib_ref = refs[:5]
        outs = refs[5:]
        dob_v, dib_v = dob_ref[...], dib_ref[...]
        u = dob_v * (p_ref[0] + p_ref[1]) - dib_v * (q_ref[0] + q_ref[1])
        x = y_ref[...] - GAMMA * u
        outs[0][...] = x
        if want_xs:
            outs[1][...] = dob_v * x

    n_out = 2 if want_xs else 1
    return pl.pallas_call(
        body,
        grid=(NP // _NB,),
        in_specs=[_nspec(), _nspec(True), _nspec(True), _nspec(), _nspec()],
        out_specs=[_nspec()] * n_out,
        out_shape=[_nshape()] * n_out,
    )(y, P, Q, dob, dib)


@jax.jit
def kernel(feat, edge_index, W1, b1, W2, b2):
    src = edge_index[0]
    dst = edge_index[1]
    pad = jnp.full((EP - E,), N, I32)
    src_p = jnp.concatenate([src, pad]).reshape(EP // SUB, SUB)
    dst_p = jnp.concatenate([dst, pad]).reshape(EP // SUB, SUB)

    h = _mlp(feat, W1, b1, W2, b2)
    h_pad = jnp.pad(h, ((0, NP - N), (0, 0)))

    dego, degi = _deg_kernel()(src_p, dst_p)
    dob, dib, xs = _prep(dego, degi, h_pad)

    pass_a = _pass_a_kernel()
    pass_b_first = _pass_b_kernel(read_z=False, write_z=True)
    pass_b_mid = _pass_b_kernel(read_z=True, write_z=True)
    pass_b_last = _pass_b_kernel(read_z=True, write_z=False)

    A = pass_a(xs, src_p, dst_p)
    y, xbs, xbd = _node1(A, h_pad, dob, dib)
    z, P, Q = pass_b_first(xbs, xbd, src_p, dst_p)
    x, xs = _node2(y, P, Q, dob, dib, want_xs=True)

    A = pass_a(xs, src_p, dst_p)
    y, xbs, xbd = _node1(A, h_pad, dob, dib, P, Q)
    z, P, Q = pass_b_mid(xbs, xbd, src_p, dst_p, z)
    x, xs = _node2(y, P, Q, dob, dib, want_xs=True)

    A = pass_a(xs, src_p, dst_p)
    y, xbs, xbd = _node1(A, h_pad, dob, dib, P, Q)
    P, Q = pass_b_last(xbs, xbd, src_p, dst_p, z)
    (x,) = _node2(y, P, Q, dob, dib, want_xs=False)

    return x[:N]
